# Initial kernel scaffold; baseline (speedup 1.0000x reference)
import jax
import jax.numpy as jnp
from jax.experimental import pallas as pl


def kernel(gt_backbone_pos, single_mask, single_res_rel, aatype, condition_mask, params):
    raise NotImplementedError("write your pallas kernel here")


# speedup vs baseline: 5.4085x; 5.4085x over previous
import functools

import jax
import jax.numpy as jnp
import numpy as np
from jax import lax
from jax.experimental import pallas as pl
from jax.experimental.pallas import tpu as pltpu
from jax.experimental.pallas import tpu_sc as plsc

B, L, K = 2, 1024, 30
NS, NV = 256, 64
ES, EV = 32, 1
ED = 512
NL = 3
EPS = 1e-8
N = B * L
E = N * K

EBN = 8
EB = EBN * K
GRID_E = N // EBN
NB = 256
GRID_N = N // NB

PD = 512
NW = 32
CHUNK = 128

_f32 = jnp.float32


def _dot(a, b):
    return jnp.dot(a, b, preferred_element_type=_f32)


@functools.partial(jax.jit, static_argnums=(2, 3))
def _sc_gather(table, idx, n_rows, d):
    per_w = n_rows // NW
    n_chunks = per_w // CHUNK
    mesh = plsc.VectorSubcoreMesh(core_axis_name="c", subcore_axis_name="s")

    @functools.partial(
        pl.kernel,
        out_type=jax.ShapeDtypeStruct((n_rows, d), _f32),
        mesh=mesh,
        scratch_types=[
            pltpu.VMEM((per_w,), jnp.int32),
            pltpu.VMEM((CHUNK, d), _f32),
            pltpu.SemaphoreType.DMA,
        ],
    )
    def gk(table_hbm, idx_hbm, out_hbm, idx_v, rows_v, gsem):
        wid = lax.axis_index("s") * 2 + lax.axis_index("c")
        base = pl.multiple_of(wid * per_w, CHUNK)
        pltpu.sync_copy(idx_hbm.at[pl.ds(base, per_w)], idx_v)

        def body(i, carry):
            off = pl.multiple_of(i * CHUNK, CHUNK)
            pltpu.async_copy(
                table_hbm.at[idx_v.at[pl.ds(off, CHUNK)]], rows_v, gsem
            ).wait()
            pltpu.sync_copy(rows_v, out_hbm.at[pl.ds(base + off, CHUNK)])
            return carry

        lax.fori_loop(0, n_chunks, body, 0)

    return gk(table, idx)


def _gvp_block(s, Vx, Vy, Vz, Wh, Wss, Wsv, bs, Wv, Wg, bg, activate):
    Vhx, Vhy, Vhz = _dot(Vx, Wh), _dot(Vy, Wh), _dot(Vz, Wh)
    vn = jnp.sqrt(Vhx * Vhx + Vhy * Vhy + Vhz * Vhz + EPS)
    so = _dot(s, Wss) + _dot(vn, Wsv) + bs
    sa = jnp.maximum(so, 0.0) if activate else so
    gate = jax.nn.sigmoid(_dot(sa, Wg) + bg)
    return sa, _dot(Vhx, Wv) * gate, _dot(Vhy, Wv) * gate, _dot(Vhz, Wv) * gate


def _ln_sv_block(s, Vx, Vy, Vz):
    mu = jnp.mean(s, axis=1, keepdims=True)
    var = jnp.mean((s - mu) * (s - mu), axis=1, keepdims=True)
    s = (s - mu) / jnp.sqrt(var + 1e-5)
    vn2 = Vx * Vx + Vy * Vy + Vz * Vz
    inv = 1.0 / jnp.sqrt(jnp.mean(vn2, axis=1, keepdims=True) + EPS)
    return s, Vx * inv, Vy * inv, Vz * inv


EEB = 960


def _edge_embed_body(esr, evr, *rest):
    os_ref, ov_ref = rest[-2], rest[-1]
    wh00, wss, wsv, bs, wv00, wg, bg = [r[...] for r in rest[:-2]]
    ev = evr[...]
    vh = ev * wh00
    vn = jnp.sqrt(vh[:, 0:1] ** 2 + vh[:, 1:2] ** 2 + vh[:, 2:3] ** 2 + EPS)
    so = _dot(esr[...], wss) + vn * wsv + bs
    gate = jax.nn.sigmoid(_dot(so, wg) + bg)
    vo = vh * (wv00 * gate)
    mu = jnp.mean(so, axis=1, keepdims=True)
    var = jnp.mean((so - mu) * (so - mu), axis=1, keepdims=True)
    os_ref[...] = (so - mu) / jnp.sqrt(var + 1e-5)
    vn2 = vo[:, 0:1] ** 2 + vo[:, 1:2] ** 2 + vo[:, 2:3] ** 2
    ov_ref[...] = vo / jnp.sqrt(vn2 + EPS)


def _edge_embed(edge_s, edge_v, p):
    grid = E // EEB
    wh00 = p['Wh'].reshape(1, 1)
    wv00 = p['Wv'].reshape(1, 1)
    wss = p['Ws'][:ES]
    wsv = p['Ws'][ES:ES + 1]
    bs = p['bs'].reshape(1, ES)
    wg = p['Wg']
    bg = p['bg'].reshape(1, 1)
    const = lambda shp: pl.BlockSpec(shp, lambda i: (0, 0))
    return pl.pallas_call(
        _edge_embed_body,
        grid=(grid,),
        in_specs=[
            pl.BlockSpec((EEB, ES), lambda i: (i, 0)),
            pl.BlockSpec((EEB, 8), lambda i: (i, 0)),
            const((1, 1)), const((ES, ES)), const((1, ES)), const((1, ES)),
            const((1, 1)), const((ES, 1)), const((1, 1)),
        ],
        out_specs=[
            pl.BlockSpec((EEB, ES), lambda i: (i, 0)),
            pl.BlockSpec((EEB, 8), lambda i: (i, 0)),
        ],
        out_shape=[
            jax.ShapeDtypeStruct((E, ES), _f32),
            jax.ShapeDtypeStruct((E, 8), _f32),
        ],
    )(edge_s, edge_v, wh00, wss, wsv, bs, wv00, wg, bg)


def _edge_msg_body(g, es, ev, hsvd, *rest):
    os_ref, ov_ref = rest[-2], rest[-1]
    (whs, whe, whd, wss0, wse0, wsd0, wsv0, bs0, wv0, wg0, bg0,
     wh1, wss1, wsv1, bs1, wv1, wg1, bg1,
     wh2, wss2, wsv2, bs2, wv2, wg2, bg2) = [r[...] for r in rest[:-2]]
    gg = g[...]
    gs = gg[:, :NS]
    gv = [gg[:, NS + 64 * c:NS + 64 * (c + 1)] for c in range(3)]
    hd = hsvd[...]
    hs_dn = hd[:, :NS]
    hv_dn = [hd[:, NS + 64 * c:NS + 64 * (c + 1)] for c in range(3)]
    rid = lax.broadcasted_iota(jnp.int32, (EB, EBN), 0) // K
    cid = lax.broadcasted_iota(jnp.int32, (EB, EBN), 1)
    bc = (rid == cid).astype(_f32)
    rid2 = lax.broadcasted_iota(jnp.int32, (EBN, EB), 0)
    cid2 = lax.broadcasted_iota(jnp.int32, (EBN, EB), 1) // K
    pm = (rid2 == cid2).astype(_f32) * (1.0 / K)

    hs_d = _dot(bc, hs_dn)
    hvdx = _dot(bc, hv_dn[0])
    hvdy = _dot(bc, hv_dn[1])
    hvdz = _dot(bc, hv_dn[2])

    evv = ev[...]
    Vhx = _dot(gv[0], whs) + evv[:, 0:1] * whe + _dot(hvdx, whd)
    Vhy = _dot(gv[1], whs) + evv[:, 1:2] * whe + _dot(hvdy, whd)
    Vhz = _dot(gv[2], whs) + evv[:, 2:3] * whe + _dot(hvdz, whd)
    vn = jnp.sqrt(Vhx * Vhx + Vhy * Vhy + Vhz * Vhz + EPS)
    so = (_dot(gs, wss0) + _dot(es[...], wse0) + _dot(hs_d, wsd0)
          + _dot(vn, wsv0) + bs0)
    s = jnp.maximum(so, 0.0)
    gate = jax.nn.sigmoid(_dot(s, wg0) + bg0)
    Vx = _dot(Vhx, wv0) * gate
    Vy = _dot(Vhy, wv0) * gate
    Vz = _dot(Vhz, wv0) * gate

    s, Vx, Vy, Vz = _gvp_block(s, Vx, Vy, Vz, wh1, wss1, wsv1, bs1, wv1, wg1,
                               bg1, True)
    s, Vx, Vy, Vz = _gvp_block(s, Vx, Vy, Vz, wh2, wss2, wsv2, bs2, wv2, wg2,
                               bg2, False)

    os_ref[...] = _dot(pm, s)
    ov_ref[0] = _dot(pm, Vx)
    ov_ref[1] = _dot(pm, Vy)
    ov_ref[2] = _dot(pm, Vz)


def _edge_msg(g, es, ev, hsv, lp):
    m0, m1, m2 = lp['m0'], lp['m1'], lp['m2']
    H0 = 2 * NV + EV
    w = [
        m0['Wh'][:NV], m0['Wh'][NV:NV + 1], m0['Wh'][NV + 1:],
        m0['Ws'][:NS], m0['Ws'][NS:NS + ES], m0['Ws'][NS + ES:2 * NS + ES],
        m0['Ws'][2 * NS + ES:], m0['bs'].reshape(1, NS), m0['Wv'],
        m0['Wg'], m0['bg'].reshape(1, NV),
    ]
    for m in (m1, m2):
        w += [m['Wh'], m['Ws'][:NS], m['Ws'][NS:], m['bs'].reshape(1, NS),
              m['Wv'], m['Wg'], m['bg'].reshape(1, NV)]
    c2 = lambda shp: pl.BlockSpec(shp, lambda i: (0, 0))
    wspecs = [c2(x.shape) for x in w]
    return pl.pallas_call(
        _edge_msg_body,
        grid=(GRID_E,),
        in_specs=[
            pl.BlockSpec((EB, PD), lambda i: (i, 0)),
            pl.BlockSpec((EB, ES), lambda i: (i, 0)),
            pl.BlockSpec((EB, 8), lambda i: (i, 0)),
            pl.BlockSpec((EBN, PD), lambda i: (i, 0)),
        ] + wspecs,
        out_specs=[
            pl.BlockSpec((EBN, NS), lambda i: (i, 0)),
            pl.BlockSpec((3, EBN, NV), lambda i: (0, i, 0)),
        ],
        out_shape=[
            jax.ShapeDtypeStruct((N, NS), _f32),
            jax.ShapeDtypeStruct((3, N, NV), _f32),
        ],
    )(g, es, ev, hsv, *w)


def _node_upd_body(hsv, ags, agv, *rest):
    o_ref = rest[-1]
    (whf0, wssf0, wsvf0, bsf0, wvf0, wgf0, bgf0,
     whf1, wssf1, wsvf1, bsf1, wvf1, wgf1, bgf1) = [r[...] for r in rest[:-1]]
    h = hsv[...]
    s = h[:, :NS] + ags[...]
    Vx = h[:, NS:NS + NV] + agv[0]
    Vy = h[:, NS + NV:NS + 2 * NV] + agv[1]
    Vz = h[:, NS + 2 * NV:NS + 3 * NV] + agv[2]
    s, Vx, Vy, Vz = _ln_sv_block(s, Vx, Vy, Vz)
    fs, fVx, fVy, fVz = _gvp_block(s, Vx, Vy, Vz, whf0, wssf0, wsvf0, bsf0,
                                   wvf0, wgf0, bgf0, True)
    fs, fVx, fVy, fVz = _gvp_block(fs, fVx, fVy, fVz, whf1, wssf1, wsvf1, bsf1,
                                   wvf1, wgf1, bgf1, False)
    s, Vx, Vy, Vz = _ln_sv_block(s + fs, Vx + fVx, Vy + fVy, Vz + fVz)
    o_ref[...] = jnp.concatenate(
        [s, Vx, Vy, Vz, jnp.zeros((s.shape[0], PD - NS - 3 * NV), _f32)],
        axis=1)


def _node_upd(hsv, ags, agv, lp):
    w = []
    for m in (lp['f0'], lp['f1']):
        w += [m['Wh'], m['Ws'][:NS], m['Ws'][NS:], m['bs'].reshape(1, NS),
              m['Wv'], m['Wg'], m['bg'].reshape(1, NV)]
    c2 = lambda shp: pl.BlockSpec(shp, lambda i: (0, 0))
    wspecs = [c2(x.shape) for x in w]
    return pl.pallas_call(
        _node_upd_body,
        grid=(GRID_N,),
        in_specs=[
            pl.BlockSpec((NB, PD), lambda i: (i, 0)),
            pl.BlockSpec((NB, NS), lambda i: (i, 0)),
            pl.BlockSpec((3, NB, NV), lambda i: (0, i, 0)),
        ] + wspecs,
        out_specs=pl.BlockSpec((NB, PD), lambda i: (i, 0)),
        out_shape=jax.ShapeDtypeStruct((N, PD), _f32),
    )(hsv, ags, agv, *w)


def _out_proj_body(hsv, rm, base, msk, *rest):
    out_ref = rest[-1]
    w0, w_0, w_1, w_2 = [r[...] for r in rest[:-1]]
    h = hsv[...]
    acc = _dot(h[:, :NS], w0) + base[...]
    Vx = h[:, NS:NS + NV]
    Vy = h[:, NS + NV:NS + 2 * NV]
    Vz = h[:, NS + 2 * NV:NS + 3 * NV]
    r = rm[...]
    wj = (w_0, w_1, w_2)
    for j in range(3):
        rot = (Vx * r[:, 3 * j:3 * j + 1] + Vy * r[:, 3 * j + 1:3 * j + 2]
               + Vz * r[:, 3 * j + 2:3 * j + 3])
        acc = acc + _dot(rot, wj[j])
    out_ref[...] = acc * msk[:, 0:1]


def _out_proj(hsv, rm, base, msk, w_out):
    w0 = w_out[:NS]
    ws = [w_out[NS + j::3] for j in range(3)]
    c2 = lambda shp: pl.BlockSpec(shp, lambda i: (0, 0))
    return pl.pallas_call(
        _out_proj_body,
        grid=(GRID_N,),
        in_specs=[
            pl.BlockSpec((NB, PD), lambda i: (i, 0)),
            pl.BlockSpec((NB, 16), lambda i: (i, 0)),
            pl.BlockSpec((NB, ED), lambda i: (i, 0)),
            pl.BlockSpec((NB, 8), lambda i: (i, 0)),
            c2(w0.shape), c2(ws[0].shape), c2(ws[1].shape), c2(ws[2].shape),
        ],
        out_specs=pl.BlockSpec((NB, ED), lambda i: (i, 0)),
        out_shape=jax.ShapeDtypeStruct((N, ED), _f32),
    )(hsv, rm, base, msk, w0, *ws)


def _norm_j(v, axis=-1, keepdims=False):
    return jnp.sqrt(jnp.sum(v * v, axis=axis, keepdims=keepdims) + EPS)


def _normalize_j(v, axis=-1):
    return v / _norm_j(v, axis=axis, keepdims=True)


def _gvp_apply_j(p, s, V, activate=True):
    Vh = jnp.einsum('...vi,vh->...hi', V, p['Wh'])
    vn = _norm_j(Vh, axis=-1)
    s_out = jnp.concatenate([s, vn], axis=-1) @ p['Ws'] + p['bs']
    Vo = jnp.einsum('...hi,hv->...vi', Vh, p['Wv'])
    gate = (jax.nn.relu(s_out) if activate else s_out) @ p['Wg'] + p['bg']
    Vo = Vo * jax.nn.sigmoid(gate)[..., None]
    if activate:
        s_out = jax.nn.relu(s_out)
    return s_out, Vo


def _ln_sv_j(s, V):
    mu = jnp.mean(s, axis=-1, keepdims=True)
    var = jnp.var(s, axis=-1, keepdims=True)
    s = (s - mu) / jnp.sqrt(var + 1e-5)
    vn2 = jnp.sum(V * V, axis=-1)
    denom = jnp.sqrt(jnp.mean(vn2, axis=-1, keepdims=True) + EPS)[..., None]
    return s, V / denom


def _ln_j(x):
    mu = jnp.mean(x, axis=-1, keepdims=True)
    var = jnp.var(x, axis=-1, keepdims=True)
    return (x - mu) / jnp.sqrt(var + 1e-5)


def kernel(gt_backbone_pos, single_mask, single_res_rel, aatype, condition_mask, params):
    X = gt_backbone_pos * condition_mask[..., None, None]
    N_, CA, C = X[..., 0, :], X[..., 1, :], X[..., 2, :]
    Xf = X.reshape(B, 3 * L, 3)
    dX = Xf[:, 1:] - Xf[:, :-1]
    U = _normalize_j(dX)
    u2, u1, u0 = U[:, :-2], U[:, 1:-1], U[:, 2:]
    n2 = _normalize_j(jnp.cross(u2, u1))
    n1 = _normalize_j(jnp.cross(u1, u0))
    cosD = jnp.clip(jnp.sum(n2 * n1, -1), -1 + 1e-7, 1 - 1e-7)
    D = jnp.sign(jnp.sum(u2 * n1, -1)) * jnp.arccos(cosD)
    D = jnp.pad(D, ((0, 0), (1, 2)))
    D = D.reshape(B, L, 3)
    dih = jnp.concatenate([jnp.cos(D), jnp.sin(D)], axis=-1)
    v1 = C - CA
    v2 = N_ - CA
    e1 = _normalize_j(v1)
    u2r = v2 - e1 * jnp.sum(e1 * v2, -1, keepdims=True)
    e2 = _normalize_j(u2r)
    e3 = jnp.cross(e1, e2)
    R = jnp.stack([e1, e2, e3], axis=-2)
    fwd = _normalize_j(jnp.pad(CA[:, 1:] - CA[:, :-1], ((0, 0), (0, 1), (0, 0))))
    bwd = _normalize_j(jnp.pad(CA[:, :-1] - CA[:, 1:], ((0, 0), (1, 0), (0, 0))))
    nv_ = _normalize_j(N_ - CA)
    cv_ = _normalize_j(C - CA)
    bis = _normalize_j(nv_ + cv_)
    perp = _normalize_j(jnp.cross(nv_, cv_))
    side = -bis * np.sqrt(1.0 / 3.0) - perp * np.sqrt(2.0 / 3.0)
    node_V = jnp.stack([fwd, bwd, side], axis=-2)
    d2 = jnp.sum((CA[:, :, None, :] - CA[:, None, :, :]) ** 2, -1)
    d2 = d2 + jnp.eye(L, dtype=_f32)[None] * 1e10
    negd, eidx = jax.lax.top_k(-d2, K)
    dist = jnp.sqrt(jnp.maximum(-negd, 0.0) + EPS)
    mu_r = jnp.linspace(0.0, 20.0, 16)
    rbf = jnp.exp(-(((dist[..., None] - mu_r) / (20.0 / 16.0)) ** 2))
    res = single_res_rel.astype(jnp.int32)
    res_j = jnp.take_along_axis(res, eidx.reshape(B, -1), axis=1).reshape(B, L, K)
    off = (res_j - res[:, :, None]).astype(_f32)
    freqs = jnp.exp(-np.log(10000.0) * jnp.arange(8) / 8.0)
    ang = off[..., None] * freqs
    pe = jnp.concatenate([jnp.cos(ang), jnp.sin(ang)], axis=-1)
    edge_s = jnp.concatenate([rbf, pe], axis=-1).reshape(E, ES)
    CA_j = jnp.take_along_axis(
        CA, eidx.reshape(B, L * K, 1), axis=1).reshape(B, L, K, 3)
    edge_v3 = _normalize_j(CA_j - CA[:, :, None, :]).reshape(E, 3)
    edge_v = jnp.pad(edge_v3, ((0, 0), (0, 5)))

    hs0, hV0 = _gvp_apply_j(params['ng'], dih, node_V, activate=False)
    hs0, hV0 = _ln_sv_j(hs0, hV0)
    hv0f = hV0.reshape(N, NV, 3)
    hsv = jnp.concatenate(
        [hs0.reshape(N, NS), hv0f[:, :, 0], hv0f[:, :, 1], hv0f[:, :, 2],
         jnp.zeros((N, PD - NS - 3 * NV), _f32)], axis=1)

    es_, ev_ = _edge_embed(edge_s, edge_v, params['eg'])

    offs = (jnp.arange(B, dtype=jnp.int32) * L)[:, None, None]
    src = (eidx.astype(jnp.int32) + offs).reshape(-1)

    for l in range(NL):
        lp = params['layers'][l]
        g = _sc_gather(hsv, src, E, PD)
        ags, agv = _edge_msg(g, es_, ev_, hsv, lp)
        hsv = _node_upd(hsv, ags, agv, lp)

    Rt_flat = R.reshape(N, 9)
    rm = jnp.pad(Rt_flat, ((0, 0), (0, 7)))
    rot_in = jnp.einsum('blvi,blij->blvj', node_V, jnp.swapaxes(R, -2, -1))
    in_feat = jnp.concatenate([dih, rot_in.reshape(B, L, 9)], axis=-1)
    comp_in = (in_feat @ params['w_in'] + params['b_in']) * np.sqrt(ED)
    conf = jnp.ones((B, L), _f32)
    mu_c = jnp.linspace(0.0, 1.0, 16)
    rbf_c = jnp.exp(-(((conf[..., None] - mu_c) * 16.0) ** 2))
    comp_conf = rbf_c @ params['w_conf'] + params['b_conf']
    comp_dih = _ln_j(dih @ params['w_dih'] + params['b_dih'])
    comp_aa = jnp.broadcast_to(params['aa_emb'][0], (B, L, ED))
    base = (comp_in + comp_dih + comp_conf + comp_aa
            + params['b_out']).reshape(N, ED)
    msk = jnp.pad(single_mask.reshape(N, 1), ((0, 0), (0, 7)))
    out = _out_proj(hsv, rm, base, msk, params['w_out'])
    return out.reshape(B, L, ED)


# speedup vs baseline: 8.1377x; 1.5046x over previous
import functools

import jax
import jax.numpy as jnp
import numpy as np
from jax import lax
from jax.experimental import pallas as pl
from jax.experimental.pallas import tpu as pltpu
from jax.experimental.pallas import tpu_sc as plsc

B, L, K = 2, 1024, 30
NS, NV = 256, 64
ES, EV = 32, 1
ED = 512
NL = 3
EPS = 1e-8
N = B * L
E = N * K

EBN = 8
EB = EBN * K
GRID_E = N // EBN
NB = 256
GRID_N = N // NB

PD = 512
NW = 32
CHUNK = 128

_f32 = jnp.float32


def _dot(a, b):
    return jnp.dot(a, b, preferred_element_type=_f32)


@functools.partial(jax.jit, static_argnums=(2, 3))
def _sc_gather(table, idx, n_rows, d):
    per_w = n_rows // NW
    n_chunks = per_w // CHUNK
    mesh = plsc.VectorSubcoreMesh(core_axis_name="c", subcore_axis_name="s")

    @functools.partial(
        pl.kernel,
        out_type=jax.ShapeDtypeStruct((n_rows, d), _f32),
        mesh=mesh,
        scratch_types=[
            pltpu.VMEM((per_w,), jnp.int32),
            pltpu.VMEM((CHUNK, d), _f32),
            pltpu.SemaphoreType.DMA,
        ],
    )
    def gk(table_hbm, idx_hbm, out_hbm, idx_v, rows_v, gsem):
        wid = lax.axis_index("s") * 2 + lax.axis_index("c")
        base = pl.multiple_of(wid * per_w, CHUNK)
        pltpu.sync_copy(idx_hbm.at[pl.ds(base, per_w)], idx_v)

        def body(i, carry):
            off = pl.multiple_of(i * CHUNK, CHUNK)
            pltpu.async_copy(
                table_hbm.at[idx_v.at[pl.ds(off, CHUNK)]], rows_v, gsem
            ).wait()
            pltpu.sync_copy(rows_v, out_hbm.at[pl.ds(base + off, CHUNK)])
            return carry

        lax.fori_loop(0, n_chunks, body, 0)

    return gk(table, idx)


def _gvp_block(s, Vx, Vy, Vz, Wh, Wss, Wsv, bs, Wv, Wg, bg, activate):
    Vhx, Vhy, Vhz = _dot(Vx, Wh), _dot(Vy, Wh), _dot(Vz, Wh)
    vn = jnp.sqrt(Vhx * Vhx + Vhy * Vhy + Vhz * Vhz + EPS)
    so = _dot(s, Wss) + _dot(vn, Wsv) + bs
    sa = jnp.maximum(so, 0.0) if activate else so
    gate = jax.nn.sigmoid(_dot(sa, Wg) + bg)
    return sa, _dot(Vhx, Wv) * gate, _dot(Vhy, Wv) * gate, _dot(Vhz, Wv) * gate


def _ln_sv_block(s, Vx, Vy, Vz):
    mu = jnp.mean(s, axis=1, keepdims=True)
    var = jnp.mean((s - mu) * (s - mu), axis=1, keepdims=True)
    s = (s - mu) / jnp.sqrt(var + 1e-5)
    vn2 = Vx * Vx + Vy * Vy + Vz * Vz
    inv = 1.0 / jnp.sqrt(jnp.mean(vn2, axis=1, keepdims=True) + EPS)
    return s, Vx * inv, Vy * inv, Vz * inv


EEB = 960
EBN2 = 32


def _edge_embed_body(nd, gp, ntab, *rest):
    os_ref, ov_ref = rest[-2], rest[-1]
    wh00, wss, wsv, bs, wv00, wg, bg = [r[...] for r in rest[:-2]]
    rid = lax.broadcasted_iota(jnp.int32, (EEB, EBN2), 0) // K
    cid = lax.broadcasted_iota(jnp.int32, (EEB, EBN2), 1)
    bc = (rid == cid).astype(_f32)
    nt = _dot(bc, ntab[...][:, :8])
    gpp = gp[...]
    res_j = gpp[:, 0:1]
    res_i = nt[:, 0:1]
    dist = jnp.sqrt(jnp.maximum(-nd[...][:, 0:1], 0.0) + EPS)
    mu_r = (lax.broadcasted_iota(jnp.int32, (1, 16), 1).astype(_f32)
            * (20.0 / 15.0))
    rbf = jnp.exp(-(((dist - mu_r) * (16.0 / 20.0)) ** 2))
    off = res_j - res_i
    freqs = jnp.exp(lax.broadcasted_iota(jnp.int32, (1, 8), 1).astype(_f32)
                    * (-np.log(10000.0) / 8.0))
    ang = off * freqs
    pe = jnp.concatenate([jnp.cos(ang), jnp.sin(ang)], axis=1)
    es = jnp.concatenate([rbf, pe], axis=1)
    dv = gpp[:, 1:4] - nt[:, 1:4]
    dvn = dv / jnp.sqrt(jnp.sum(dv * dv, axis=1, keepdims=True) + EPS)
    ev = jnp.concatenate([dvn, jnp.zeros((EEB, 5), _f32)], axis=1)
    vh = ev * wh00
    vn = jnp.sqrt(vh[:, 0:1] ** 2 + vh[:, 1:2] ** 2 + vh[:, 2:3] ** 2 + EPS)
    so = _dot(es, wss) + vn * wsv + bs
    gate = jax.nn.sigmoid(_dot(so, wg) + bg)
    vo = vh * (wv00 * gate)
    mu = jnp.mean(so, axis=1, keepdims=True)
    var = jnp.mean((so - mu) * (so - mu), axis=1, keepdims=True)
    os_ref[...] = (so - mu) / jnp.sqrt(var + 1e-5)
    vn2 = vo[:, 0:1] ** 2 + vo[:, 1:2] ** 2 + vo[:, 2:3] ** 2
    ov_ref[...] = vo / jnp.sqrt(vn2 + EPS)


def _edge_embed(nd, gp, ntab, p):
    grid = E // EEB
    wh00 = p['Wh'].reshape(1, 1)
    wv00 = p['Wv'].reshape(1, 1)
    wss = p['Ws'][:ES]
    wsv = p['Ws'][ES:ES + 1]
    bs = p['bs'].reshape(1, ES)
    wg = p['Wg']
    bg = p['bg'].reshape(1, 1)
    const = lambda shp: pl.BlockSpec(shp, lambda i: (0, 0))
    return pl.pallas_call(
        _edge_embed_body,
        grid=(grid,),
        in_specs=[
            pl.BlockSpec((EEB, 8), lambda i: (i, 0)),
            pl.BlockSpec((EEB, 128), lambda i: (i, 0)),
            pl.BlockSpec((EBN2, 128), lambda i: (i, 0)),
            const((1, 1)), const((ES, ES)), const((1, ES)), const((1, ES)),
            const((1, 1)), const((ES, 1)), const((1, 1)),
        ],
        out_specs=[
            pl.BlockSpec((EEB, ES), lambda i: (i, 0)),
            pl.BlockSpec((EEB, 8), lambda i: (i, 0)),
        ],
        out_shape=[
            jax.ShapeDtypeStruct((E, ES), _f32),
            jax.ShapeDtypeStruct((E, 8), _f32),
        ],
    )(nd, gp, ntab, wh00, wss, wsv, bs, wv00, wg, bg)


def _edge_msg_body(g, es, ev, hsvd, *rest):
    os_ref, ov_ref = rest[-2], rest[-1]
    (whs, whe, whd, wss0, wse0, wsd0, wsv0, bs0, wv0, wg0, bg0,
     wh1, wss1, wsv1, bs1, wv1, wg1, bg1,
     wh2, wss2, wsv2, bs2, wv2, wg2, bg2) = [r[...] for r in rest[:-2]]
    gg = g[...]
    gs = gg[:, :NS]
    gv = [gg[:, NS + 64 * c:NS + 64 * (c + 1)] for c in range(3)]
    hd = hsvd[...]
    hs_dn = hd[:, :NS]
    hv_dn = [hd[:, NS + 64 * c:NS + 64 * (c + 1)] for c in range(3)]
    rid = lax.broadcasted_iota(jnp.int32, (EB, EBN), 0) // K
    cid = lax.broadcasted_iota(jnp.int32, (EB, EBN), 1)
    bc = (rid == cid).astype(_f32)
    rid2 = lax.broadcasted_iota(jnp.int32, (EBN, EB), 0)
    cid2 = lax.broadcasted_iota(jnp.int32, (EBN, EB), 1) // K
    pm = (rid2 == cid2).astype(_f32) * (1.0 / K)

    hs_d = _dot(bc, hs_dn)
    hvdx = _dot(bc, hv_dn[0])
    hvdy = _dot(bc, hv_dn[1])
    hvdz = _dot(bc, hv_dn[2])

    evv = ev[...]
    Vhx = _dot(gv[0], whs) + evv[:, 0:1] * whe + _dot(hvdx, whd)
    Vhy = _dot(gv[1], whs) + evv[:, 1:2] * whe + _dot(hvdy, whd)
    Vhz = _dot(gv[2], whs) + evv[:, 2:3] * whe + _dot(hvdz, whd)
    vn = jnp.sqrt(Vhx * Vhx + Vhy * Vhy + Vhz * Vhz + EPS)
    so = (_dot(gs, wss0) + _dot(es[...], wse0) + _dot(hs_d, wsd0)
          + _dot(vn, wsv0) + bs0)
    s = jnp.maximum(so, 0.0)
    gate = jax.nn.sigmoid(_dot(s, wg0) + bg0)
    Vx = _dot(Vhx, wv0) * gate
    Vy = _dot(Vhy, wv0) * gate
    Vz = _dot(Vhz, wv0) * gate

    s, Vx, Vy, Vz = _gvp_block(s, Vx, Vy, Vz, wh1, wss1, wsv1, bs1, wv1, wg1,
                               bg1, True)
    s, Vx, Vy, Vz = _gvp_block(s, Vx, Vy, Vz, wh2, wss2, wsv2, bs2, wv2, wg2,
                               bg2, False)

    os_ref[...] = _dot(pm, s)
    ov_ref[0] = _dot(pm, Vx)
    ov_ref[1] = _dot(pm, Vy)
    ov_ref[2] = _dot(pm, Vz)


def _edge_msg(g, es, ev, hsv, lp):
    m0, m1, m2 = lp['m0'], lp['m1'], lp['m2']
    H0 = 2 * NV + EV
    w = [
        m0['Wh'][:NV], m0['Wh'][NV:NV + 1], m0['Wh'][NV + 1:],
        m0['Ws'][:NS], m0['Ws'][NS:NS + ES], m0['Ws'][NS + ES:2 * NS + ES],
        m0['Ws'][2 * NS + ES:], m0['bs'].reshape(1, NS), m0['Wv'],
        m0['Wg'], m0['bg'].reshape(1, NV),
    ]
    for m in (m1, m2):
        w += [m['Wh'], m['Ws'][:NS], m['Ws'][NS:], m['bs'].reshape(1, NS),
              m['Wv'], m['Wg'], m['bg'].reshape(1, NV)]
    c2 = lambda shp: pl.BlockSpec(shp, lambda i: (0, 0))
    wspecs = [c2(x.shape) for x in w]
    return pl.pallas_call(
        _edge_msg_body,
        grid=(GRID_E,),
        in_specs=[
            pl.BlockSpec((EB, PD), lambda i: (i, 0)),
            pl.BlockSpec((EB, ES), lambda i: (i, 0)),
            pl.BlockSpec((EB, 8), lambda i: (i, 0)),
            pl.BlockSpec((EBN, PD), lambda i: (i, 0)),
        ] + wspecs,
        out_specs=[
            pl.BlockSpec((EBN, NS), lambda i: (i, 0)),
            pl.BlockSpec((3, EBN, NV), lambda i: (0, i, 0)),
        ],
        out_shape=[
            jax.ShapeDtypeStruct((N, NS), _f32),
            jax.ShapeDtypeStruct((3, N, NV), _f32),
        ],
    )(g, es, ev, hsv, *w)


def _node_upd_body(hsv, ags, agv, *rest):
    o_ref = rest[-1]
    (whf0, wssf0, wsvf0, bsf0, wvf0, wgf0, bgf0,
     whf1, wssf1, wsvf1, bsf1, wvf1, wgf1, bgf1) = [r[...] for r in rest[:-1]]
    h = hsv[...]
    s = h[:, :NS] + ags[...]
    Vx = h[:, NS:NS + NV] + agv[0]
    Vy = h[:, NS + NV:NS + 2 * NV] + agv[1]
    Vz = h[:, NS + 2 * NV:NS + 3 * NV] + agv[2]
    s, Vx, Vy, Vz = _ln_sv_block(s, Vx, Vy, Vz)
    fs, fVx, fVy, fVz = _gvp_block(s, Vx, Vy, Vz, whf0, wssf0, wsvf0, bsf0,
                                   wvf0, wgf0, bgf0, True)
    fs, fVx, fVy, fVz = _gvp_block(fs, fVx, fVy, fVz, whf1, wssf1, wsvf1, bsf1,
                                   wvf1, wgf1, bgf1, False)
    s, Vx, Vy, Vz = _ln_sv_block(s + fs, Vx + fVx, Vy + fVy, Vz + fVz)
    o_ref[...] = jnp.concatenate(
        [s, Vx, Vy, Vz, jnp.zeros((s.shape[0], PD - NS - 3 * NV), _f32)],
        axis=1)


def _node_upd(hsv, ags, agv, lp):
    w = []
    for m in (lp['f0'], lp['f1']):
        w += [m['Wh'], m['Ws'][:NS], m['Ws'][NS:], m['bs'].reshape(1, NS),
              m['Wv'], m['Wg'], m['bg'].reshape(1, NV)]
    c2 = lambda shp: pl.BlockSpec(shp, lambda i: (0, 0))
    wspecs = [c2(x.shape) for x in w]
    return pl.pallas_call(
        _node_upd_body,
        grid=(GRID_N,),
        in_specs=[
            pl.BlockSpec((NB, PD), lambda i: (i, 0)),
            pl.BlockSpec((NB, NS), lambda i: (i, 0)),
            pl.BlockSpec((3, NB, NV), lambda i: (0, i, 0)),
        ] + wspecs,
        out_specs=pl.BlockSpec((NB, PD), lambda i: (i, 0)),
        out_shape=jax.ShapeDtypeStruct((N, PD), _f32),
    )(hsv, ags, agv, *w)


def _out_proj_body(hsv, rm, base, msk, *rest):
    out_ref = rest[-1]
    w0, w_0, w_1, w_2 = [r[...] for r in rest[:-1]]
    h = hsv[...]
    acc = _dot(h[:, :NS], w0) + base[...]
    Vx = h[:, NS:NS + NV]
    Vy = h[:, NS + NV:NS + 2 * NV]
    Vz = h[:, NS + 2 * NV:NS + 3 * NV]
    r = rm[...]
    wj = (w_0, w_1, w_2)
    for j in range(3):
        rot = (Vx * r[:, 3 * j:3 * j + 1] + Vy * r[:, 3 * j + 1:3 * j + 2]
               + Vz * r[:, 3 * j + 2:3 * j + 3])
        acc = acc + _dot(rot, wj[j])
    out_ref[...] = acc * msk[:, 0:1]


def _out_proj(hsv, rm, base, msk, w_out):
    w0 = w_out[:NS]
    ws = [w_out[NS + j::3] for j in range(3)]
    c2 = lambda shp: pl.BlockSpec(shp, lambda i: (0, 0))
    return pl.pallas_call(
        _out_proj_body,
        grid=(GRID_N,),
        in_specs=[
            pl.BlockSpec((NB, PD), lambda i: (i, 0)),
            pl.BlockSpec((NB, 16), lambda i: (i, 0)),
            pl.BlockSpec((NB, ED), lambda i: (i, 0)),
            pl.BlockSpec((NB, 8), lambda i: (i, 0)),
            c2(w0.shape), c2(ws[0].shape), c2(ws[1].shape), c2(ws[2].shape),
        ],
        out_specs=pl.BlockSpec((NB, ED), lambda i: (i, 0)),
        out_shape=jax.ShapeDtypeStruct((N, ED), _f32),
    )(hsv, rm, base, msk, w0, *ws)


def _norm_j(v, axis=-1, keepdims=False):
    return jnp.sqrt(jnp.sum(v * v, axis=axis, keepdims=keepdims) + EPS)


def _normalize_j(v, axis=-1):
    return v / _norm_j(v, axis=axis, keepdims=True)


def _gvp_apply_j(p, s, V, activate=True):
    Vh = jnp.einsum('...vi,vh->...hi', V, p['Wh'])
    vn = _norm_j(Vh, axis=-1)
    s_out = jnp.concatenate([s, vn], axis=-1) @ p['Ws'] + p['bs']
    Vo = jnp.einsum('...hi,hv->...vi', Vh, p['Wv'])
    gate = (jax.nn.relu(s_out) if activate else s_out) @ p['Wg'] + p['bg']
    Vo = Vo * jax.nn.sigmoid(gate)[..., None]
    if activate:
        s_out = jax.nn.relu(s_out)
    return s_out, Vo


def _ln_sv_j(s, V):
    mu = jnp.mean(s, axis=-1, keepdims=True)
    var = jnp.var(s, axis=-1, keepdims=True)
    s = (s - mu) / jnp.sqrt(var + 1e-5)
    vn2 = jnp.sum(V * V, axis=-1)
    denom = jnp.sqrt(jnp.mean(vn2, axis=-1, keepdims=True) + EPS)[..., None]
    return s, V / denom


def _ln_j(x):
    mu = jnp.mean(x, axis=-1, keepdims=True)
    var = jnp.var(x, axis=-1, keepdims=True)
    return (x - mu) / jnp.sqrt(var + 1e-5)


def kernel(gt_backbone_pos, single_mask, single_res_rel, aatype, condition_mask, params):
    X = gt_backbone_pos * condition_mask[..., None, None]
    N_, CA, C = X[..., 0, :], X[..., 1, :], X[..., 2, :]
    Xf = X.reshape(B, 3 * L, 3)
    dX = Xf[:, 1:] - Xf[:, :-1]
    U = _normalize_j(dX)
    u2, u1, u0 = U[:, :-2], U[:, 1:-1], U[:, 2:]
    n2 = _normalize_j(jnp.cross(u2, u1))
    n1 = _normalize_j(jnp.cross(u1, u0))
    cosD = jnp.clip(jnp.sum(n2 * n1, -1), -1 + 1e-7, 1 - 1e-7)
    D = jnp.sign(jnp.sum(u2 * n1, -1)) * jnp.arccos(cosD)
    D = jnp.pad(D, ((0, 0), (1, 2)))
    D = D.reshape(B, L, 3)
    dih = jnp.concatenate([jnp.cos(D), jnp.sin(D)], axis=-1)
    v1 = C - CA
    v2 = N_ - CA
    e1 = _normalize_j(v1)
    u2r = v2 - e1 * jnp.sum(e1 * v2, -1, keepdims=True)
    e2 = _normalize_j(u2r)
    e3 = jnp.cross(e1, e2)
    R = jnp.stack([e1, e2, e3], axis=-2)
    fwd = _normalize_j(jnp.pad(CA[:, 1:] - CA[:, :-1], ((0, 0), (0, 1), (0, 0))))
    bwd = _normalize_j(jnp.pad(CA[:, :-1] - CA[:, 1:], ((0, 0), (1, 0), (0, 0))))
    nv_ = _normalize_j(N_ - CA)
    cv_ = _normalize_j(C - CA)
    bis = _normalize_j(nv_ + cv_)
    perp = _normalize_j(jnp.cross(nv_, cv_))
    side = -bis * np.sqrt(1.0 / 3.0) - perp * np.sqrt(2.0 / 3.0)
    node_V = jnp.stack([fwd, bwd, side], axis=-2)
    d2 = jnp.sum((CA[:, :, None, :] - CA[:, None, :, :]) ** 2, -1)
    d2 = d2 + jnp.eye(L, dtype=_f32)[None] * 1e10
    negd, eidx = jax.lax.top_k(-d2, K)
    res_f = single_res_rel.astype(_f32).reshape(N, 1)
    ntab = jnp.concatenate(
        [res_f, CA.reshape(N, 3), jnp.zeros((N, 124), _f32)], axis=1)
    nd = jnp.pad(negd.reshape(E, 1), ((0, 0), (0, 7)))

    hs0, hV0 = _gvp_apply_j(params['ng'], dih, node_V, activate=False)
    hs0, hV0 = _ln_sv_j(hs0, hV0)
    hv0f = hV0.reshape(N, NV, 3)
    hsv = jnp.concatenate(
        [hs0.reshape(N, NS), hv0f[:, :, 0], hv0f[:, :, 1], hv0f[:, :, 2],
         jnp.zeros((N, PD - NS - 3 * NV), _f32)], axis=1)

    offs = (jnp.arange(B, dtype=jnp.int32) * L)[:, None, None]
    src = (eidx.astype(jnp.int32) + offs).reshape(-1)

    gp = _sc_gather(ntab, src, E, 128)
    es_, ev_ = _edge_embed(nd, gp, ntab, params['eg'])

    for l in range(NL):
        lp = params['layers'][l]
        g = _sc_gather(hsv, src, E, PD)
        ags, agv = _edge_msg(g, es_, ev_, hsv, lp)
        hsv = _node_upd(hsv, ags, agv, lp)

    Rt_flat = R.reshape(N, 9)
    rm = jnp.pad(Rt_flat, ((0, 0), (0, 7)))
    rot_in = jnp.einsum('blvi,blij->blvj', node_V, jnp.swapaxes(R, -2, -1))
    in_feat = jnp.concatenate([dih, rot_in.reshape(B, L, 9)], axis=-1)
    comp_in = (in_feat @ params['w_in'] + params['b_in']) * np.sqrt(ED)
    conf = jnp.ones((B, L), _f32)
    mu_c = jnp.linspace(0.0, 1.0, 16)
    rbf_c = jnp.exp(-(((conf[..., None] - mu_c) * 16.0) ** 2))
    comp_conf = rbf_c @ params['w_conf'] + params['b_conf']
    comp_dih = _ln_j(dih @ params['w_dih'] + params['b_dih'])
    comp_aa = jnp.broadcast_to(params['aa_emb'][0], (B, L, ED))
    base = (comp_in + comp_dih + comp_conf + comp_aa
            + params['b_out']).reshape(N, ED)
    msk = jnp.pad(single_mask.reshape(N, 1), ((0, 0), (0, 7)))
    out = _out_proj(hsv, rm, base, msk, params['w_out'])
    return out.reshape(B, L, ED)


# speedup vs baseline: 9.4167x; 1.1572x over previous
import functools

import jax
import jax.numpy as jnp
import numpy as np
from jax import lax
from jax.experimental import pallas as pl
from jax.experimental.pallas import tpu as pltpu
from jax.experimental.pallas import tpu_sc as plsc

B, L, K = 2, 1024, 30
NS, NV = 256, 64
ES, EV = 32, 1
ED = 512
NL = 3
EPS = 1e-8
N = B * L
E = N * K

EBN = 16
EB = EBN * K
GRID_E = N // EBN
NB = 256
GRID_N = N // NB

PD = 512
NW = 32
CHUNK = 128

_f32 = jnp.float32


def _dot(a, b):
    return jnp.dot(a, b, preferred_element_type=_f32)


@functools.partial(jax.jit, static_argnums=(2, 3))
def _sc_gather(table, idx, n_rows, d):
    per_w = n_rows // NW
    n_chunks = per_w // CHUNK
    mesh = plsc.VectorSubcoreMesh(core_axis_name="c", subcore_axis_name="s")

    @functools.partial(
        pl.kernel,
        out_type=jax.ShapeDtypeStruct((n_rows, d), _f32),
        mesh=mesh,
        scratch_types=[
            pltpu.VMEM((per_w,), jnp.int32),
            pltpu.VMEM((CHUNK, d), _f32),
            pltpu.SemaphoreType.DMA,
        ],
    )
    def gk(table_hbm, idx_hbm, out_hbm, idx_v, rows_v, gsem):
        wid = lax.axis_index("s") * 2 + lax.axis_index("c")
        base = pl.multiple_of(wid * per_w, CHUNK)
        pltpu.sync_copy(idx_hbm.at[pl.ds(base, per_w)], idx_v)

        def body(i, carry):
            off = pl.multiple_of(i * CHUNK, CHUNK)
            pltpu.async_copy(
                table_hbm.at[idx_v.at[pl.ds(off, CHUNK)]], rows_v, gsem
            ).wait()
            pltpu.sync_copy(rows_v, out_hbm.at[pl.ds(base + off, CHUNK)])
            return carry

        lax.fori_loop(0, n_chunks, body, 0)

    return gk(table, idx)


def _gvp_block(s, Vx, Vy, Vz, Wh, Wss, Wsv, bs, Wv, Wg, bg, activate):
    Vhx, Vhy, Vhz = _dot(Vx, Wh), _dot(Vy, Wh), _dot(Vz, Wh)
    vn = jnp.sqrt(Vhx * Vhx + Vhy * Vhy + Vhz * Vhz + EPS)
    so = _dot(s, Wss) + _dot(vn, Wsv) + bs
    sa = jnp.maximum(so, 0.0) if activate else so
    gate = jax.nn.sigmoid(_dot(sa, Wg) + bg)
    return sa, _dot(Vhx, Wv) * gate, _dot(Vhy, Wv) * gate, _dot(Vhz, Wv) * gate


def _ln_sv_block(s, Vx, Vy, Vz):
    mu = jnp.mean(s, axis=1, keepdims=True)
    var = jnp.mean((s - mu) * (s - mu), axis=1, keepdims=True)
    s = (s - mu) / jnp.sqrt(var + 1e-5)
    vn2 = Vx * Vx + Vy * Vy + Vz * Vz
    inv = 1.0 / jnp.sqrt(jnp.mean(vn2, axis=1, keepdims=True) + EPS)
    return s, Vx * inv, Vy * inv, Vz * inv


EEB = 960
EBN2 = 32


def _edge_embed_body(nd, gp, ntab, *rest):
    os_ref, ov_ref = rest[-2], rest[-1]
    wh00, wss, wsv, bs, wv00, wg, bg = [r[...] for r in rest[:-2]]
    rid = lax.broadcasted_iota(jnp.int32, (EEB, EBN2), 0) // K
    cid = lax.broadcasted_iota(jnp.int32, (EEB, EBN2), 1)
    bc = (rid == cid).astype(_f32)
    nt = _dot(bc, ntab[...][:, :8])
    gpp = gp[...]
    res_j = gpp[:, 0:1]
    res_i = nt[:, 0:1]
    dist = jnp.sqrt(jnp.maximum(-nd[...][:, 0:1], 0.0) + EPS)
    mu_r = (lax.broadcasted_iota(jnp.int32, (1, 16), 1).astype(_f32)
            * (20.0 / 15.0))
    rbf = jnp.exp(-(((dist - mu_r) * (16.0 / 20.0)) ** 2))
    off = res_j - res_i
    freqs = jnp.exp(lax.broadcasted_iota(jnp.int32, (1, 8), 1).astype(_f32)
                    * (-np.log(10000.0) / 8.0))
    ang = off * freqs
    pe = jnp.concatenate([jnp.cos(ang), jnp.sin(ang)], axis=1)
    es = jnp.concatenate([rbf, pe], axis=1)
    dv = gpp[:, 1:4] - nt[:, 1:4]
    dvn = dv / jnp.sqrt(jnp.sum(dv * dv, axis=1, keepdims=True) + EPS)
    ev = jnp.concatenate([dvn, jnp.zeros((EEB, 5), _f32)], axis=1)
    vh = ev * wh00
    vn = jnp.sqrt(vh[:, 0:1] ** 2 + vh[:, 1:2] ** 2 + vh[:, 2:3] ** 2 + EPS)
    so = _dot(es, wss) + vn * wsv + bs
    gate = jax.nn.sigmoid(_dot(so, wg) + bg)
    vo = vh * (wv00 * gate)
    mu = jnp.mean(so, axis=1, keepdims=True)
    var = jnp.mean((so - mu) * (so - mu), axis=1, keepdims=True)
    os_ref[...] = (so - mu) / jnp.sqrt(var + 1e-5)
    vn2 = vo[:, 0:1] ** 2 + vo[:, 1:2] ** 2 + vo[:, 2:3] ** 2
    ov_ref[...] = vo / jnp.sqrt(vn2 + EPS)


def _edge_embed(nd, gp, ntab, p):
    grid = E // EEB
    wh00 = p['Wh'].reshape(1, 1)
    wv00 = p['Wv'].reshape(1, 1)
    wss = p['Ws'][:ES]
    wsv = p['Ws'][ES:ES + 1]
    bs = p['bs'].reshape(1, ES)
    wg = p['Wg']
    bg = p['bg'].reshape(1, 1)
    const = lambda shp: pl.BlockSpec(shp, lambda i: (0, 0))
    return pl.pallas_call(
        _edge_embed_body,
        grid=(grid,),
        in_specs=[
            pl.BlockSpec((EEB, 8), lambda i: (i, 0)),
            pl.BlockSpec((EEB, 128), lambda i: (i, 0)),
            pl.BlockSpec((EBN2, 128), lambda i: (i, 0)),
            const((1, 1)), const((ES, ES)), const((1, ES)), const((1, ES)),
            const((1, 1)), const((ES, 1)), const((1, 1)),
        ],
        out_specs=[
            pl.BlockSpec((EEB, ES), lambda i: (i, 0)),
            pl.BlockSpec((EEB, 8), lambda i: (i, 0)),
        ],
        out_shape=[
            jax.ShapeDtypeStruct((E, ES), _f32),
            jax.ShapeDtypeStruct((E, 8), _f32),
        ],
    )(nd, gp, ntab, wh00, wss, wsv, bs, wv00, wg, bg)


def _edge_msg_body(g, es, ev, hsvd, *rest):
    os_ref, ov_ref = rest[-2], rest[-1]
    (whs, whe, whd, wss0, wse0, wsd0, wsv0, bs0, wv0, wg0, bg0,
     wh1, wss1, wsv1, bs1, wv1, wg1, bg1,
     wh2, wss2, wsv2, bs2, wv2, wg2, bg2) = [r[...] for r in rest[:-2]]
    gg = g[...]
    gs = gg[:, :NS]
    gv = [gg[:, NS + 64 * c:NS + 64 * (c + 1)] for c in range(3)]
    hd = hsvd[...]
    hs_dn = hd[:, :NS]
    hv_dn = [hd[:, NS + 64 * c:NS + 64 * (c + 1)] for c in range(3)]
    rid = lax.broadcasted_iota(jnp.int32, (EB, EBN), 0) // K
    cid = lax.broadcasted_iota(jnp.int32, (EB, EBN), 1)
    bc = (rid == cid).astype(_f32)
    rid2 = lax.broadcasted_iota(jnp.int32, (EBN, EB), 0)
    cid2 = lax.broadcasted_iota(jnp.int32, (EBN, EB), 1) // K
    pm = (rid2 == cid2).astype(_f32) * (1.0 / K)

    hs_d = _dot(bc, hs_dn)
    hvdx = _dot(bc, hv_dn[0])
    hvdy = _dot(bc, hv_dn[1])
    hvdz = _dot(bc, hv_dn[2])

    evv = ev[...]
    Vhx = _dot(gv[0], whs) + evv[:, 0:1] * whe + _dot(hvdx, whd)
    Vhy = _dot(gv[1], whs) + evv[:, 1:2] * whe + _dot(hvdy, whd)
    Vhz = _dot(gv[2], whs) + evv[:, 2:3] * whe + _dot(hvdz, whd)
    vn = jnp.sqrt(Vhx * Vhx + Vhy * Vhy + Vhz * Vhz + EPS)
    so = (_dot(gs, wss0) + _dot(es[...], wse0) + _dot(hs_d, wsd0)
          + _dot(vn, wsv0) + bs0)
    s = jnp.maximum(so, 0.0)
    gate = jax.nn.sigmoid(_dot(s, wg0) + bg0)
    Vx = _dot(Vhx, wv0) * gate
    Vy = _dot(Vhy, wv0) * gate
    Vz = _dot(Vhz, wv0) * gate

    s, Vx, Vy, Vz = _gvp_block(s, Vx, Vy, Vz, wh1, wss1, wsv1, bs1, wv1, wg1,
                               bg1, True)
    s, Vx, Vy, Vz = _gvp_block(s, Vx, Vy, Vz, wh2, wss2, wsv2, bs2, wv2, wg2,
                               bg2, False)

    os_ref[...] = _dot(pm, s)
    ov_ref[0] = _dot(pm, Vx)
    ov_ref[1] = _dot(pm, Vy)
    ov_ref[2] = _dot(pm, Vz)


def _edge_msg(g, es, ev, hsv, lp):
    m0, m1, m2 = lp['m0'], lp['m1'], lp['m2']
    H0 = 2 * NV + EV
    w = [
        m0['Wh'][:NV], m0['Wh'][NV:NV + 1], m0['Wh'][NV + 1:],
        m0['Ws'][:NS], m0['Ws'][NS:NS + ES], m0['Ws'][NS + ES:2 * NS + ES],
        m0['Ws'][2 * NS + ES:], m0['bs'].reshape(1, NS), m0['Wv'],
        m0['Wg'], m0['bg'].reshape(1, NV),
    ]
    for m in (m1, m2):
        w += [m['Wh'], m['Ws'][:NS], m['Ws'][NS:], m['bs'].reshape(1, NS),
              m['Wv'], m['Wg'], m['bg'].reshape(1, NV)]
    c2 = lambda shp: pl.BlockSpec(shp, lambda i: (0, 0))
    wspecs = [c2(x.shape) for x in w]
    return pl.pallas_call(
        _edge_msg_body,
        grid=(GRID_E,),
        in_specs=[
            pl.BlockSpec((EB, PD), lambda i: (i, 0)),
            pl.BlockSpec((EB, ES), lambda i: (i, 0)),
            pl.BlockSpec((EB, 8), lambda i: (i, 0)),
            pl.BlockSpec((EBN, PD), lambda i: (i, 0)),
        ] + wspecs,
        out_specs=[
            pl.BlockSpec((EBN, NS), lambda i: (i, 0)),
            pl.BlockSpec((3, EBN, NV), lambda i: (0, i, 0)),
        ],
        out_shape=[
            jax.ShapeDtypeStruct((N, NS), _f32),
            jax.ShapeDtypeStruct((3, N, NV), _f32),
        ],
    )(g, es, ev, hsv, *w)


def _node_upd_body(hsv, ags, agv, *rest):
    o_ref = rest[-1]
    (whf0, wssf0, wsvf0, bsf0, wvf0, wgf0, bgf0,
     whf1, wssf1, wsvf1, bsf1, wvf1, wgf1, bgf1) = [r[...] for r in rest[:-1]]
    h = hsv[...]
    s = h[:, :NS] + ags[...]
    Vx = h[:, NS:NS + NV] + agv[0]
    Vy = h[:, NS + NV:NS + 2 * NV] + agv[1]
    Vz = h[:, NS + 2 * NV:NS + 3 * NV] + agv[2]
    s, Vx, Vy, Vz = _ln_sv_block(s, Vx, Vy, Vz)
    fs, fVx, fVy, fVz = _gvp_block(s, Vx, Vy, Vz, whf0, wssf0, wsvf0, bsf0,
                                   wvf0, wgf0, bgf0, True)
    fs, fVx, fVy, fVz = _gvp_block(fs, fVx, fVy, fVz, whf1, wssf1, wsvf1, bsf1,
                                   wvf1, wgf1, bgf1, False)
    s, Vx, Vy, Vz = _ln_sv_block(s + fs, Vx + fVx, Vy + fVy, Vz + fVz)
    o_ref[...] = jnp.concatenate(
        [s, Vx, Vy, Vz, jnp.zeros((s.shape[0], PD - NS - 3 * NV), _f32)],
        axis=1)


def _node_upd(hsv, ags, agv, lp):
    w = []
    for m in (lp['f0'], lp['f1']):
        w += [m['Wh'], m['Ws'][:NS], m['Ws'][NS:], m['bs'].reshape(1, NS),
              m['Wv'], m['Wg'], m['bg'].reshape(1, NV)]
    c2 = lambda shp: pl.BlockSpec(shp, lambda i: (0, 0))
    wspecs = [c2(x.shape) for x in w]
    return pl.pallas_call(
        _node_upd_body,
        grid=(GRID_N,),
        in_specs=[
            pl.BlockSpec((NB, PD), lambda i: (i, 0)),
            pl.BlockSpec((NB, NS), lambda i: (i, 0)),
            pl.BlockSpec((3, NB, NV), lambda i: (0, i, 0)),
        ] + wspecs,
        out_specs=pl.BlockSpec((NB, PD), lambda i: (i, 0)),
        out_shape=jax.ShapeDtypeStruct((N, PD), _f32),
    )(hsv, ags, agv, *w)


def _out_proj_body(hsv, rm, base, msk, *rest):
    out_ref = rest[-1]
    w0, w_0, w_1, w_2 = [r[...] for r in rest[:-1]]
    h = hsv[...]
    acc = _dot(h[:, :NS], w0) + base[...]
    Vx = h[:, NS:NS + NV]
    Vy = h[:, NS + NV:NS + 2 * NV]
    Vz = h[:, NS + 2 * NV:NS + 3 * NV]
    r = rm[...]
    wj = (w_0, w_1, w_2)
    for j in range(3):
        rot = (Vx * r[:, 3 * j:3 * j + 1] + Vy * r[:, 3 * j + 1:3 * j + 2]
               + Vz * r[:, 3 * j + 2:3 * j + 3])
        acc = acc + _dot(rot, wj[j])
    out_ref[...] = acc * msk[:, 0:1]


def _out_proj(hsv, rm, base, msk, w_out):
    w0 = w_out[:NS]
    ws = [w_out[NS + j::3] for j in range(3)]
    c2 = lambda shp: pl.BlockSpec(shp, lambda i: (0, 0))
    return pl.pallas_call(
        _out_proj_body,
        grid=(GRID_N,),
        in_specs=[
            pl.BlockSpec((NB, PD), lambda i: (i, 0)),
            pl.BlockSpec((NB, 16), lambda i: (i, 0)),
            pl.BlockSpec((NB, ED), lambda i: (i, 0)),
            pl.BlockSpec((NB, 8), lambda i: (i, 0)),
            c2(w0.shape), c2(ws[0].shape), c2(ws[1].shape), c2(ws[2].shape),
        ],
        out_specs=pl.BlockSpec((NB, ED), lambda i: (i, 0)),
        out_shape=jax.ShapeDtypeStruct((N, ED), _f32),
    )(hsv, rm, base, msk, w0, *ws)


def _norm_j(v, axis=-1, keepdims=False):
    return jnp.sqrt(jnp.sum(v * v, axis=axis, keepdims=keepdims) + EPS)


def _normalize_j(v, axis=-1):
    return v / _norm_j(v, axis=axis, keepdims=True)


def _gvp_apply_j(p, s, V, activate=True):
    Vh = jnp.einsum('...vi,vh->...hi', V, p['Wh'])
    vn = _norm_j(Vh, axis=-1)
    s_out = jnp.concatenate([s, vn], axis=-1) @ p['Ws'] + p['bs']
    Vo = jnp.einsum('...hi,hv->...vi', Vh, p['Wv'])
    gate = (jax.nn.relu(s_out) if activate else s_out) @ p['Wg'] + p['bg']
    Vo = Vo * jax.nn.sigmoid(gate)[..., None]
    if activate:
        s_out = jax.nn.relu(s_out)
    return s_out, Vo


def _ln_sv_j(s, V):
    mu = jnp.mean(s, axis=-1, keepdims=True)
    var = jnp.var(s, axis=-1, keepdims=True)
    s = (s - mu) / jnp.sqrt(var + 1e-5)
    vn2 = jnp.sum(V * V, axis=-1)
    denom = jnp.sqrt(jnp.mean(vn2, axis=-1, keepdims=True) + EPS)[..., None]
    return s, V / denom


def _ln_j(x):
    mu = jnp.mean(x, axis=-1, keepdims=True)
    var = jnp.var(x, axis=-1, keepdims=True)
    return (x - mu) / jnp.sqrt(var + 1e-5)


def kernel(gt_backbone_pos, single_mask, single_res_rel, aatype, condition_mask, params):
    X = gt_backbone_pos * condition_mask[..., None, None]
    N_, CA, C = X[..., 0, :], X[..., 1, :], X[..., 2, :]
    Xf = X.reshape(B, 3 * L, 3)
    dX = Xf[:, 1:] - Xf[:, :-1]
    U = _normalize_j(dX)
    u2, u1, u0 = U[:, :-2], U[:, 1:-1], U[:, 2:]
    n2 = _normalize_j(jnp.cross(u2, u1))
    n1 = _normalize_j(jnp.cross(u1, u0))
    cosD = jnp.clip(jnp.sum(n2 * n1, -1), -1 + 1e-7, 1 - 1e-7)
    D = jnp.sign(jnp.sum(u2 * n1, -1)) * jnp.arccos(cosD)
    D = jnp.pad(D, ((0, 0), (1, 2)))
    D = D.reshape(B, L, 3)
    dih = jnp.concatenate([jnp.cos(D), jnp.sin(D)], axis=-1)
    v1 = C - CA
    v2 = N_ - CA
    e1 = _normalize_j(v1)
    u2r = v2 - e1 * jnp.sum(e1 * v2, -1, keepdims=True)
    e2 = _normalize_j(u2r)
    e3 = jnp.cross(e1, e2)
    R = jnp.stack([e1, e2, e3], axis=-2)
    fwd = _normalize_j(jnp.pad(CA[:, 1:] - CA[:, :-1], ((0, 0), (0, 1), (0, 0))))
    bwd = _normalize_j(jnp.pad(CA[:, :-1] - CA[:, 1:], ((0, 0), (1, 0), (0, 0))))
    nv_ = _normalize_j(N_ - CA)
    cv_ = _normalize_j(C - CA)
    bis = _normalize_j(nv_ + cv_)
    perp = _normalize_j(jnp.cross(nv_, cv_))
    side = -bis * np.sqrt(1.0 / 3.0) - perp * np.sqrt(2.0 / 3.0)
    node_V = jnp.stack([fwd, bwd, side], axis=-2)
    d2 = jnp.sum((CA[:, :, None, :] - CA[:, None, :, :]) ** 2, -1)
    d2 = d2 + jnp.eye(L, dtype=_f32)[None] * 1e10
    negd, eidx = jax.lax.top_k(-d2, K)
    res_f = single_res_rel.astype(_f32).reshape(N, 1)
    ntab = jnp.concatenate(
        [res_f, CA.reshape(N, 3), jnp.zeros((N, 124), _f32)], axis=1)
    nd = jnp.pad(negd.reshape(E, 1), ((0, 0), (0, 7)))

    hs0, hV0 = _gvp_apply_j(params['ng'], dih, node_V, activate=False)
    hs0, hV0 = _ln_sv_j(hs0, hV0)
    hv0f = hV0.reshape(N, NV, 3)
    hsv = jnp.concatenate(
        [hs0.reshape(N, NS), hv0f[:, :, 0], hv0f[:, :, 1], hv0f[:, :, 2],
         jnp.zeros((N, PD - NS - 3 * NV), _f32)], axis=1)

    offs = (jnp.arange(B, dtype=jnp.int32) * L)[:, None, None]
    src = (eidx.astype(jnp.int32) + offs).reshape(-1)

    gp = _sc_gather(ntab, src, E, 128)
    es_, ev_ = _edge_embed(nd, gp, ntab, params['eg'])

    for l in range(NL):
        lp = params['layers'][l]
        g = _sc_gather(hsv, src, E, PD)
        ags, agv = _edge_msg(g, es_, ev_, hsv, lp)
        hsv = _node_upd(hsv, ags, agv, lp)

    Rt_flat = R.reshape(N, 9)
    rm = jnp.pad(Rt_flat, ((0, 0), (0, 7)))
    rot_in = jnp.einsum('blvi,blij->blvj', node_V, jnp.swapaxes(R, -2, -1))
    in_feat = jnp.concatenate([dih, rot_in.reshape(B, L, 9)], axis=-1)
    comp_in = (in_feat @ params['w_in'] + params['b_in']) * np.sqrt(ED)
    conf = jnp.ones((B, L), _f32)
    mu_c = jnp.linspace(0.0, 1.0, 16)
    rbf_c = jnp.exp(-(((conf[..., None] - mu_c) * 16.0) ** 2))
    comp_conf = rbf_c @ params['w_conf'] + params['b_conf']
    comp_dih = _ln_j(dih @ params['w_dih'] + params['b_dih'])
    comp_aa = jnp.broadcast_to(params['aa_emb'][0], (B, L, ED))
    base = (comp_in + comp_dih + comp_conf + comp_aa
            + params['b_out']).reshape(N, ED)
    msk = jnp.pad(single_mask.reshape(N, 1), ((0, 0), (0, 7)))
    out = _out_proj(hsv, rm, base, msk, params['w_out'])
    return out.reshape(B, L, ED)


# speedup vs baseline: 10.2692x; 1.0905x over previous
import functools

import jax
import jax.numpy as jnp
import numpy as np
from jax import lax
from jax.experimental import pallas as pl
from jax.experimental.pallas import tpu as pltpu
from jax.experimental.pallas import tpu_sc as plsc

B, L, K = 2, 1024, 30
NS, NV = 256, 64
ES, EV = 32, 1
ED = 512
NL = 3
EPS = 1e-8
N = B * L
E = N * K

EBN = 32
EB = EBN * K
GRID_E = N // EBN
NB = 256
GRID_N = N // NB

PD = 512
NW = 32
CHUNK = 128

_f32 = jnp.float32


def _dot(a, b):
    return jnp.dot(a, b, preferred_element_type=_f32)


@functools.partial(jax.jit, static_argnums=(2, 3))
def _sc_gather(table, idx, n_rows, d):
    per_w = n_rows // NW
    n_chunks = per_w // CHUNK
    mesh = plsc.VectorSubcoreMesh(core_axis_name="c", subcore_axis_name="s")

    @functools.partial(
        pl.kernel,
        out_type=jax.ShapeDtypeStruct((n_rows, d), _f32),
        mesh=mesh,
        scratch_types=[
            pltpu.VMEM((per_w,), jnp.int32),
            pltpu.VMEM((CHUNK, d), _f32),
            pltpu.SemaphoreType.DMA,
        ],
    )
    def gk(table_hbm, idx_hbm, out_hbm, idx_v, rows_v, gsem):
        wid = lax.axis_index("s") * 2 + lax.axis_index("c")
        base = pl.multiple_of(wid * per_w, CHUNK)
        pltpu.sync_copy(idx_hbm.at[pl.ds(base, per_w)], idx_v)

        def body(i, carry):
            off = pl.multiple_of(i * CHUNK, CHUNK)
            pltpu.async_copy(
                table_hbm.at[idx_v.at[pl.ds(off, CHUNK)]], rows_v, gsem
            ).wait()
            pltpu.sync_copy(rows_v, out_hbm.at[pl.ds(base + off, CHUNK)])
            return carry

        lax.fori_loop(0, n_chunks, body, 0)

    return gk(table, idx)


def _gvp_block(s, Vx, Vy, Vz, Wh, Wss, Wsv, bs, Wv, Wg, bg, activate):
    Vhx, Vhy, Vhz = _dot(Vx, Wh), _dot(Vy, Wh), _dot(Vz, Wh)
    vn = jnp.sqrt(Vhx * Vhx + Vhy * Vhy + Vhz * Vhz + EPS)
    so = _dot(s, Wss) + _dot(vn, Wsv) + bs
    sa = jnp.maximum(so, 0.0) if activate else so
    gate = jax.nn.sigmoid(_dot(sa, Wg) + bg)
    return sa, _dot(Vhx, Wv) * gate, _dot(Vhy, Wv) * gate, _dot(Vhz, Wv) * gate


def _ln_sv_block(s, Vx, Vy, Vz):
    mu = jnp.mean(s, axis=1, keepdims=True)
    var = jnp.mean((s - mu) * (s - mu), axis=1, keepdims=True)
    s = (s - mu) / jnp.sqrt(var + 1e-5)
    vn2 = Vx * Vx + Vy * Vy + Vz * Vz
    inv = 1.0 / jnp.sqrt(jnp.mean(vn2, axis=1, keepdims=True) + EPS)
    return s, Vx * inv, Vy * inv, Vz * inv


EEB = 960
EBN2 = 32


def _edge_embed_body(nd, gp, ntab, *rest):
    os_ref, ov_ref = rest[-2], rest[-1]
    wh00, wss, wsv, bs, wv00, wg, bg = [r[...] for r in rest[:-2]]
    rid = lax.broadcasted_iota(jnp.int32, (EEB, EBN2), 0) // K
    cid = lax.broadcasted_iota(jnp.int32, (EEB, EBN2), 1)
    bc = (rid == cid).astype(_f32)
    nt = _dot(bc, ntab[...][:, :8])
    gpp = gp[...]
    res_j = gpp[:, 0:1]
    res_i = nt[:, 0:1]
    dist = jnp.sqrt(jnp.maximum(-nd[...][:, 0:1], 0.0) + EPS)
    mu_r = (lax.broadcasted_iota(jnp.int32, (1, 16), 1).astype(_f32)
            * (20.0 / 15.0))
    rbf = jnp.exp(-(((dist - mu_r) * (16.0 / 20.0)) ** 2))
    off = res_j - res_i
    freqs = jnp.exp(lax.broadcasted_iota(jnp.int32, (1, 8), 1).astype(_f32)
                    * (-np.log(10000.0) / 8.0))
    ang = off * freqs
    pe = jnp.concatenate([jnp.cos(ang), jnp.sin(ang)], axis=1)
    es = jnp.concatenate([rbf, pe], axis=1)
    dv = gpp[:, 1:4] - nt[:, 1:4]
    dvn = dv / jnp.sqrt(jnp.sum(dv * dv, axis=1, keepdims=True) + EPS)
    ev = jnp.concatenate([dvn, jnp.zeros((EEB, 5), _f32)], axis=1)
    vh = ev * wh00
    vn = jnp.sqrt(vh[:, 0:1] ** 2 + vh[:, 1:2] ** 2 + vh[:, 2:3] ** 2 + EPS)
    so = _dot(es, wss) + vn * wsv + bs
    gate = jax.nn.sigmoid(_dot(so, wg) + bg)
    vo = vh * (wv00 * gate)
    mu = jnp.mean(so, axis=1, keepdims=True)
    var = jnp.mean((so - mu) * (so - mu), axis=1, keepdims=True)
    os_ref[...] = (so - mu) / jnp.sqrt(var + 1e-5)
    vn2 = vo[:, 0:1] ** 2 + vo[:, 1:2] ** 2 + vo[:, 2:3] ** 2
    ov_ref[...] = vo / jnp.sqrt(vn2 + EPS)


def _edge_embed(nd, gp, ntab, p):
    grid = E // EEB
    wh00 = p['Wh'].reshape(1, 1)
    wv00 = p['Wv'].reshape(1, 1)
    wss = p['Ws'][:ES]
    wsv = p['Ws'][ES:ES + 1]
    bs = p['bs'].reshape(1, ES)
    wg = p['Wg']
    bg = p['bg'].reshape(1, 1)
    const = lambda shp: pl.BlockSpec(shp, lambda i: (0, 0))
    return pl.pallas_call(
        _edge_embed_body,
        grid=(grid,),
        in_specs=[
            pl.BlockSpec((EEB, 8), lambda i: (i, 0)),
            pl.BlockSpec((EEB, 128), lambda i: (i, 0)),
            pl.BlockSpec((EBN2, 128), lambda i: (i, 0)),
            const((1, 1)), const((ES, ES)), const((1, ES)), const((1, ES)),
            const((1, 1)), const((ES, 1)), const((1, 1)),
        ],
        out_specs=[
            pl.BlockSpec((EEB, ES), lambda i: (i, 0)),
            pl.BlockSpec((EEB, 8), lambda i: (i, 0)),
        ],
        out_shape=[
            jax.ShapeDtypeStruct((E, ES), _f32),
            jax.ShapeDtypeStruct((E, 8), _f32),
        ],
    )(nd, gp, ntab, wh00, wss, wsv, bs, wv00, wg, bg)


def _edge_msg_body(g, es, ev, hsvd, *rest):
    os_ref, ov_ref = rest[-2], rest[-1]
    (whs, whe, whd, wss0, wse0, wsd0, wsv0, bs0, wv0, wg0, bg0,
     wh1, wss1, wsv1, bs1, wv1, wg1, bg1,
     wh2, wss2, wsv2, bs2, wv2, wg2, bg2) = [r[...] for r in rest[:-2]]
    gg = g[...]
    gs = gg[:, :NS]
    gv = [gg[:, NS + 64 * c:NS + 64 * (c + 1)] for c in range(3)]
    hd = hsvd[...]
    hs_dn = hd[:, :NS]
    hv_dn = [hd[:, NS + 64 * c:NS + 64 * (c + 1)] for c in range(3)]
    rid = lax.broadcasted_iota(jnp.int32, (EB, EBN), 0) // K
    cid = lax.broadcasted_iota(jnp.int32, (EB, EBN), 1)
    bc = (rid == cid).astype(_f32)
    rid2 = lax.broadcasted_iota(jnp.int32, (EBN, EB), 0)
    cid2 = lax.broadcasted_iota(jnp.int32, (EBN, EB), 1) // K
    pm = (rid2 == cid2).astype(_f32) * (1.0 / K)

    hs_d = _dot(bc, hs_dn)
    hvdx = _dot(bc, hv_dn[0])
    hvdy = _dot(bc, hv_dn[1])
    hvdz = _dot(bc, hv_dn[2])

    evv = ev[...]
    Vhx = _dot(gv[0], whs) + evv[:, 0:1] * whe + _dot(hvdx, whd)
    Vhy = _dot(gv[1], whs) + evv[:, 1:2] * whe + _dot(hvdy, whd)
    Vhz = _dot(gv[2], whs) + evv[:, 2:3] * whe + _dot(hvdz, whd)
    vn = jnp.sqrt(Vhx * Vhx + Vhy * Vhy + Vhz * Vhz + EPS)
    so = (_dot(gs, wss0) + _dot(es[...], wse0) + _dot(hs_d, wsd0)
          + _dot(vn, wsv0) + bs0)
    s = jnp.maximum(so, 0.0)
    gate = jax.nn.sigmoid(_dot(s, wg0) + bg0)
    Vx = _dot(Vhx, wv0) * gate
    Vy = _dot(Vhy, wv0) * gate
    Vz = _dot(Vhz, wv0) * gate

    s, Vx, Vy, Vz = _gvp_block(s, Vx, Vy, Vz, wh1, wss1, wsv1, bs1, wv1, wg1,
                               bg1, True)
    s, Vx, Vy, Vz = _gvp_block(s, Vx, Vy, Vz, wh2, wss2, wsv2, bs2, wv2, wg2,
                               bg2, False)

    os_ref[...] = _dot(pm, s)
    ov_ref[0] = _dot(pm, Vx)
    ov_ref[1] = _dot(pm, Vy)
    ov_ref[2] = _dot(pm, Vz)


def _edge_msg(g, es, ev, hsv, lp):
    m0, m1, m2 = lp['m0'], lp['m1'], lp['m2']
    H0 = 2 * NV + EV
    w = [
        m0['Wh'][:NV], m0['Wh'][NV:NV + 1], m0['Wh'][NV + 1:],
        m0['Ws'][:NS], m0['Ws'][NS:NS + ES], m0['Ws'][NS + ES:2 * NS + ES],
        m0['Ws'][2 * NS + ES:], m0['bs'].reshape(1, NS), m0['Wv'],
        m0['Wg'], m0['bg'].reshape(1, NV),
    ]
    for m in (m1, m2):
        w += [m['Wh'], m['Ws'][:NS], m['Ws'][NS:], m['bs'].reshape(1, NS),
              m['Wv'], m['Wg'], m['bg'].reshape(1, NV)]
    c2 = lambda shp: pl.BlockSpec(shp, lambda i: (0, 0))
    wspecs = [c2(x.shape) for x in w]
    return pl.pallas_call(
        _edge_msg_body,
        grid=(GRID_E,),
        in_specs=[
            pl.BlockSpec((EB, PD), lambda i: (i, 0)),
            pl.BlockSpec((EB, ES), lambda i: (i, 0)),
            pl.BlockSpec((EB, 8), lambda i: (i, 0)),
            pl.BlockSpec((EBN, PD), lambda i: (i, 0)),
        ] + wspecs,
        out_specs=[
            pl.BlockSpec((EBN, NS), lambda i: (i, 0)),
            pl.BlockSpec((3, EBN, NV), lambda i: (0, i, 0)),
        ],
        out_shape=[
            jax.ShapeDtypeStruct((N, NS), _f32),
            jax.ShapeDtypeStruct((3, N, NV), _f32),
        ],
    )(g, es, ev, hsv, *w)


def _node_upd_body(hsv, ags, agv, *rest):
    o_ref = rest[-1]
    (whf0, wssf0, wsvf0, bsf0, wvf0, wgf0, bgf0,
     whf1, wssf1, wsvf1, bsf1, wvf1, wgf1, bgf1) = [r[...] for r in rest[:-1]]
    h = hsv[...]
    s = h[:, :NS] + ags[...]
    Vx = h[:, NS:NS + NV] + agv[0]
    Vy = h[:, NS + NV:NS + 2 * NV] + agv[1]
    Vz = h[:, NS + 2 * NV:NS + 3 * NV] + agv[2]
    s, Vx, Vy, Vz = _ln_sv_block(s, Vx, Vy, Vz)
    fs, fVx, fVy, fVz = _gvp_block(s, Vx, Vy, Vz, whf0, wssf0, wsvf0, bsf0,
                                   wvf0, wgf0, bgf0, True)
    fs, fVx, fVy, fVz = _gvp_block(fs, fVx, fVy, fVz, whf1, wssf1, wsvf1, bsf1,
                                   wvf1, wgf1, bgf1, False)
    s, Vx, Vy, Vz = _ln_sv_block(s + fs, Vx + fVx, Vy + fVy, Vz + fVz)
    o_ref[...] = jnp.concatenate(
        [s, Vx, Vy, Vz, jnp.zeros((s.shape[0], PD - NS - 3 * NV), _f32)],
        axis=1)


def _node_upd(hsv, ags, agv, lp):
    w = []
    for m in (lp['f0'], lp['f1']):
        w += [m['Wh'], m['Ws'][:NS], m['Ws'][NS:], m['bs'].reshape(1, NS),
              m['Wv'], m['Wg'], m['bg'].reshape(1, NV)]
    c2 = lambda shp: pl.BlockSpec(shp, lambda i: (0, 0))
    wspecs = [c2(x.shape) for x in w]
    return pl.pallas_call(
        _node_upd_body,
        grid=(GRID_N,),
        in_specs=[
            pl.BlockSpec((NB, PD), lambda i: (i, 0)),
            pl.BlockSpec((NB, NS), lambda i: (i, 0)),
            pl.BlockSpec((3, NB, NV), lambda i: (0, i, 0)),
        ] + wspecs,
        out_specs=pl.BlockSpec((NB, PD), lambda i: (i, 0)),
        out_shape=jax.ShapeDtypeStruct((N, PD), _f32),
    )(hsv, ags, agv, *w)


def _out_proj_body(hsv, rm, base, msk, *rest):
    out_ref = rest[-1]
    w0, w_0, w_1, w_2 = [r[...] for r in rest[:-1]]
    h = hsv[...]
    acc = _dot(h[:, :NS], w0) + base[...]
    Vx = h[:, NS:NS + NV]
    Vy = h[:, NS + NV:NS + 2 * NV]
    Vz = h[:, NS + 2 * NV:NS + 3 * NV]
    r = rm[...]
    wj = (w_0, w_1, w_2)
    for j in range(3):
        rot = (Vx * r[:, 3 * j:3 * j + 1] + Vy * r[:, 3 * j + 1:3 * j + 2]
               + Vz * r[:, 3 * j + 2:3 * j + 3])
        acc = acc + _dot(rot, wj[j])
    out_ref[...] = acc * msk[:, 0:1]


def _out_proj(hsv, rm, base, msk, w_out):
    w0 = w_out[:NS]
    ws = [w_out[NS + j::3] for j in range(3)]
    c2 = lambda shp: pl.BlockSpec(shp, lambda i: (0, 0))
    return pl.pallas_call(
        _out_proj_body,
        grid=(GRID_N,),
        in_specs=[
            pl.BlockSpec((NB, PD), lambda i: (i, 0)),
            pl.BlockSpec((NB, 16), lambda i: (i, 0)),
            pl.BlockSpec((NB, ED), lambda i: (i, 0)),
            pl.BlockSpec((NB, 8), lambda i: (i, 0)),
            c2(w0.shape), c2(ws[0].shape), c2(ws[1].shape), c2(ws[2].shape),
        ],
        out_specs=pl.BlockSpec((NB, ED), lambda i: (i, 0)),
        out_shape=jax.ShapeDtypeStruct((N, ED), _f32),
    )(hsv, rm, base, msk, w0, *ws)


def _norm_j(v, axis=-1, keepdims=False):
    return jnp.sqrt(jnp.sum(v * v, axis=axis, keepdims=keepdims) + EPS)


def _normalize_j(v, axis=-1):
    return v / _norm_j(v, axis=axis, keepdims=True)


def _gvp_apply_j(p, s, V, activate=True):
    Vh = jnp.einsum('...vi,vh->...hi', V, p['Wh'])
    vn = _norm_j(Vh, axis=-1)
    s_out = jnp.concatenate([s, vn], axis=-1) @ p['Ws'] + p['bs']
    Vo = jnp.einsum('...hi,hv->...vi', Vh, p['Wv'])
    gate = (jax.nn.relu(s_out) if activate else s_out) @ p['Wg'] + p['bg']
    Vo = Vo * jax.nn.sigmoid(gate)[..., None]
    if activate:
        s_out = jax.nn.relu(s_out)
    return s_out, Vo


def _ln_sv_j(s, V):
    mu = jnp.mean(s, axis=-1, keepdims=True)
    var = jnp.var(s, axis=-1, keepdims=True)
    s = (s - mu) / jnp.sqrt(var + 1e-5)
    vn2 = jnp.sum(V * V, axis=-1)
    denom = jnp.sqrt(jnp.mean(vn2, axis=-1, keepdims=True) + EPS)[..., None]
    return s, V / denom


def _ln_j(x):
    mu = jnp.mean(x, axis=-1, keepdims=True)
    var = jnp.var(x, axis=-1, keepdims=True)
    return (x - mu) / jnp.sqrt(var + 1e-5)


def kernel(gt_backbone_pos, single_mask, single_res_rel, aatype, condition_mask, params):
    X = gt_backbone_pos * condition_mask[..., None, None]
    N_, CA, C = X[..., 0, :], X[..., 1, :], X[..., 2, :]
    Xf = X.reshape(B, 3 * L, 3)
    dX = Xf[:, 1:] - Xf[:, :-1]
    U = _normalize_j(dX)
    u2, u1, u0 = U[:, :-2], U[:, 1:-1], U[:, 2:]
    n2 = _normalize_j(jnp.cross(u2, u1))
    n1 = _normalize_j(jnp.cross(u1, u0))
    cosD = jnp.clip(jnp.sum(n2 * n1, -1), -1 + 1e-7, 1 - 1e-7)
    D = jnp.sign(jnp.sum(u2 * n1, -1)) * jnp.arccos(cosD)
    D = jnp.pad(D, ((0, 0), (1, 2)))
    D = D.reshape(B, L, 3)
    dih = jnp.concatenate([jnp.cos(D), jnp.sin(D)], axis=-1)
    v1 = C - CA
    v2 = N_ - CA
    e1 = _normalize_j(v1)
    u2r = v2 - e1 * jnp.sum(e1 * v2, -1, keepdims=True)
    e2 = _normalize_j(u2r)
    e3 = jnp.cross(e1, e2)
    R = jnp.stack([e1, e2, e3], axis=-2)
    fwd = _normalize_j(jnp.pad(CA[:, 1:] - CA[:, :-1], ((0, 0), (0, 1), (0, 0))))
    bwd = _normalize_j(jnp.pad(CA[:, :-1] - CA[:, 1:], ((0, 0), (1, 0), (0, 0))))
    nv_ = _normalize_j(N_ - CA)
    cv_ = _normalize_j(C - CA)
    bis = _normalize_j(nv_ + cv_)
    perp = _normalize_j(jnp.cross(nv_, cv_))
    side = -bis * np.sqrt(1.0 / 3.0) - perp * np.sqrt(2.0 / 3.0)
    node_V = jnp.stack([fwd, bwd, side], axis=-2)
    d2 = jnp.sum((CA[:, :, None, :] - CA[:, None, :, :]) ** 2, -1)
    d2 = d2 + jnp.eye(L, dtype=_f32)[None] * 1e10
    negd, eidx = jax.lax.top_k(-d2, K)
    res_f = single_res_rel.astype(_f32).reshape(N, 1)
    ntab = jnp.concatenate(
        [res_f, CA.reshape(N, 3), jnp.zeros((N, 124), _f32)], axis=1)
    nd = jnp.pad(negd.reshape(E, 1), ((0, 0), (0, 7)))

    hs0, hV0 = _gvp_apply_j(params['ng'], dih, node_V, activate=False)
    hs0, hV0 = _ln_sv_j(hs0, hV0)
    hv0f = hV0.reshape(N, NV, 3)
    hsv = jnp.concatenate(
        [hs0.reshape(N, NS), hv0f[:, :, 0], hv0f[:, :, 1], hv0f[:, :, 2],
         jnp.zeros((N, PD - NS - 3 * NV), _f32)], axis=1)

    offs = (jnp.arange(B, dtype=jnp.int32) * L)[:, None, None]
    src = (eidx.astype(jnp.int32) + offs).reshape(-1)

    gp = _sc_gather(ntab, src, E, 128)
    es_, ev_ = _edge_embed(nd, gp, ntab, params['eg'])

    for l in range(NL):
        lp = params['layers'][l]
        g = _sc_gather(hsv, src, E, PD)
        ags, agv = _edge_msg(g, es_, ev_, hsv, lp)
        hsv = _node_upd(hsv, ags, agv, lp)

    Rt_flat = R.reshape(N, 9)
    rm = jnp.pad(Rt_flat, ((0, 0), (0, 7)))
    rot_in = jnp.einsum('blvi,blij->blvj', node_V, jnp.swapaxes(R, -2, -1))
    in_feat = jnp.concatenate([dih, rot_in.reshape(B, L, 9)], axis=-1)
    comp_in = (in_feat @ params['w_in'] + params['b_in']) * np.sqrt(ED)
    conf = jnp.ones((B, L), _f32)
    mu_c = jnp.linspace(0.0, 1.0, 16)
    rbf_c = jnp.exp(-(((conf[..., None] - mu_c) * 16.0) ** 2))
    comp_conf = rbf_c @ params['w_conf'] + params['b_conf']
    comp_dih = _ln_j(dih @ params['w_dih'] + params['b_dih'])
    comp_aa = jnp.broadcast_to(params['aa_emb'][0], (B, L, ED))
    base = (comp_in + comp_dih + comp_conf + comp_aa
            + params['b_out']).reshape(N, ED)
    msk = jnp.pad(single_mask.reshape(N, 1), ((0, 0), (0, 7)))
    out = _out_proj(hsv, rm, base, msk, params['w_out'])
    return out.reshape(B, L, ED)


# speedup vs baseline: 10.5988x; 1.0321x over previous
import functools

import jax
import jax.numpy as jnp
import numpy as np
from jax import lax
from jax.experimental import pallas as pl
from jax.experimental.pallas import tpu as pltpu
from jax.experimental.pallas import tpu_sc as plsc

B, L, K = 2, 1024, 30
NS, NV = 256, 64
ES, EV = 32, 1
ED = 512
NL = 3
EPS = 1e-8
N = B * L
E = N * K

EBN = 64
EB = EBN * K
GRID_E = N // EBN
NB = 256
GRID_N = N // NB

PD = 512
NW = 32
CHUNK = 128

_f32 = jnp.float32


def _dot(a, b):
    return jnp.dot(a, b, preferred_element_type=_f32)


@functools.partial(jax.jit, static_argnums=(2, 3))
def _sc_gather(table, idx, n_rows, d):
    per_w = n_rows // NW
    n_chunks = per_w // CHUNK
    mesh = plsc.VectorSubcoreMesh(core_axis_name="c", subcore_axis_name="s")

    @functools.partial(
        pl.kernel,
        out_type=jax.ShapeDtypeStruct((n_rows, d), _f32),
        mesh=mesh,
        scratch_types=[
            pltpu.VMEM((per_w,), jnp.int32),
            pltpu.VMEM((CHUNK, d), _f32),
            pltpu.SemaphoreType.DMA,
        ],
    )
    def gk(table_hbm, idx_hbm, out_hbm, idx_v, rows_v, gsem):
        wid = lax.axis_index("s") * 2 + lax.axis_index("c")
        base = pl.multiple_of(wid * per_w, CHUNK)
        pltpu.sync_copy(idx_hbm.at[pl.ds(base, per_w)], idx_v)

        def body(i, carry):
            off = pl.multiple_of(i * CHUNK, CHUNK)
            pltpu.async_copy(
                table_hbm.at[idx_v.at[pl.ds(off, CHUNK)]], rows_v, gsem
            ).wait()
            pltpu.sync_copy(rows_v, out_hbm.at[pl.ds(base + off, CHUNK)])
            return carry

        lax.fori_loop(0, n_chunks, body, 0)

    return gk(table, idx)


def _gvp_block(s, Vx, Vy, Vz, Wh, Wss, Wsv, bs, Wv, Wg, bg, activate):
    Vhx, Vhy, Vhz = _dot(Vx, Wh), _dot(Vy, Wh), _dot(Vz, Wh)
    vn = jnp.sqrt(Vhx * Vhx + Vhy * Vhy + Vhz * Vhz + EPS)
    so = _dot(s, Wss) + _dot(vn, Wsv) + bs
    sa = jnp.maximum(so, 0.0) if activate else so
    gate = jax.nn.sigmoid(_dot(sa, Wg) + bg)
    return sa, _dot(Vhx, Wv) * gate, _dot(Vhy, Wv) * gate, _dot(Vhz, Wv) * gate


def _ln_sv_block(s, Vx, Vy, Vz):
    mu = jnp.mean(s, axis=1, keepdims=True)
    var = jnp.mean((s - mu) * (s - mu), axis=1, keepdims=True)
    s = (s - mu) / jnp.sqrt(var + 1e-5)
    vn2 = Vx * Vx + Vy * Vy + Vz * Vz
    inv = 1.0 / jnp.sqrt(jnp.mean(vn2, axis=1, keepdims=True) + EPS)
    return s, Vx * inv, Vy * inv, Vz * inv


EEB = 960
EBN2 = 32


def _edge_embed_body(nd, gp, ntab, *rest):
    os_ref, ov_ref = rest[-2], rest[-1]
    wh00, wss, wsv, bs, wv00, wg, bg = [r[...] for r in rest[:-2]]
    rid = lax.broadcasted_iota(jnp.int32, (EEB, EBN2), 0) // K
    cid = lax.broadcasted_iota(jnp.int32, (EEB, EBN2), 1)
    bc = (rid == cid).astype(_f32)
    nt = _dot(bc, ntab[...][:, :8])
    gpp = gp[...]
    res_j = gpp[:, 0:1]
    res_i = nt[:, 0:1]
    dist = jnp.sqrt(jnp.maximum(-nd[...][:, 0:1], 0.0) + EPS)
    mu_r = (lax.broadcasted_iota(jnp.int32, (1, 16), 1).astype(_f32)
            * (20.0 / 15.0))
    rbf = jnp.exp(-(((dist - mu_r) * (16.0 / 20.0)) ** 2))
    off = res_j - res_i
    freqs = jnp.exp(lax.broadcasted_iota(jnp.int32, (1, 8), 1).astype(_f32)
                    * (-np.log(10000.0) / 8.0))
    ang = off * freqs
    pe = jnp.concatenate([jnp.cos(ang), jnp.sin(ang)], axis=1)
    es = jnp.concatenate([rbf, pe], axis=1)
    dv = gpp[:, 1:4] - nt[:, 1:4]
    dvn = dv / jnp.sqrt(jnp.sum(dv * dv, axis=1, keepdims=True) + EPS)
    ev = jnp.concatenate([dvn, jnp.zeros((EEB, 5), _f32)], axis=1)
    vh = ev * wh00
    vn = jnp.sqrt(vh[:, 0:1] ** 2 + vh[:, 1:2] ** 2 + vh[:, 2:3] ** 2 + EPS)
    so = _dot(es, wss) + vn * wsv + bs
    gate = jax.nn.sigmoid(_dot(so, wg) + bg)
    vo = vh * (wv00 * gate)
    mu = jnp.mean(so, axis=1, keepdims=True)
    var = jnp.mean((so - mu) * (so - mu), axis=1, keepdims=True)
    os_ref[...] = (so - mu) / jnp.sqrt(var + 1e-5)
    vn2 = vo[:, 0:1] ** 2 + vo[:, 1:2] ** 2 + vo[:, 2:3] ** 2
    ov_ref[...] = vo / jnp.sqrt(vn2 + EPS)


def _edge_embed(nd, gp, ntab, p):
    grid = E // EEB
    wh00 = p['Wh'].reshape(1, 1)
    wv00 = p['Wv'].reshape(1, 1)
    wss = p['Ws'][:ES]
    wsv = p['Ws'][ES:ES + 1]
    bs = p['bs'].reshape(1, ES)
    wg = p['Wg']
    bg = p['bg'].reshape(1, 1)
    const = lambda shp: pl.BlockSpec(shp, lambda i: (0, 0))
    return pl.pallas_call(
        _edge_embed_body,
        grid=(grid,),
        in_specs=[
            pl.BlockSpec((EEB, 8), lambda i: (i, 0)),
            pl.BlockSpec((EEB, 128), lambda i: (i, 0)),
            pl.BlockSpec((EBN2, 128), lambda i: (i, 0)),
            const((1, 1)), const((ES, ES)), const((1, ES)), const((1, ES)),
            const((1, 1)), const((ES, 1)), const((1, 1)),
        ],
        out_specs=[
            pl.BlockSpec((EEB, ES), lambda i: (i, 0)),
            pl.BlockSpec((EEB, 8), lambda i: (i, 0)),
        ],
        out_shape=[
            jax.ShapeDtypeStruct((E, ES), _f32),
            jax.ShapeDtypeStruct((E, 8), _f32),
        ],
    )(nd, gp, ntab, wh00, wss, wsv, bs, wv00, wg, bg)


def _edge_msg_body(g, es, ev, hsvd, *rest):
    os_ref, ov_ref = rest[-2], rest[-1]
    (whs, whe, whd, wss0, wse0, wsd0, wsv0, bs0, wv0, wg0, bg0,
     wh1, wss1, wsv1, bs1, wv1, wg1, bg1,
     wh2, wss2, wsv2, bs2, wv2, wg2, bg2) = [r[...] for r in rest[:-2]]
    gg = g[...]
    gs = gg[:, :NS]
    gv = [gg[:, NS + 64 * c:NS + 64 * (c + 1)] for c in range(3)]
    hd = hsvd[...]
    hs_dn = hd[:, :NS]
    hv_dn = [hd[:, NS + 64 * c:NS + 64 * (c + 1)] for c in range(3)]
    rid = lax.broadcasted_iota(jnp.int32, (EB, EBN), 0) // K
    cid = lax.broadcasted_iota(jnp.int32, (EB, EBN), 1)
    bc = (rid == cid).astype(_f32)
    rid2 = lax.broadcasted_iota(jnp.int32, (EBN, EB), 0)
    cid2 = lax.broadcasted_iota(jnp.int32, (EBN, EB), 1) // K
    pm = (rid2 == cid2).astype(_f32) * (1.0 / K)

    hs_d = _dot(bc, hs_dn)
    hvdx = _dot(bc, hv_dn[0])
    hvdy = _dot(bc, hv_dn[1])
    hvdz = _dot(bc, hv_dn[2])

    evv = ev[...]
    Vhx = _dot(gv[0], whs) + evv[:, 0:1] * whe + _dot(hvdx, whd)
    Vhy = _dot(gv[1], whs) + evv[:, 1:2] * whe + _dot(hvdy, whd)
    Vhz = _dot(gv[2], whs) + evv[:, 2:3] * whe + _dot(hvdz, whd)
    vn = jnp.sqrt(Vhx * Vhx + Vhy * Vhy + Vhz * Vhz + EPS)
    so = (_dot(gs, wss0) + _dot(es[...], wse0) + _dot(hs_d, wsd0)
          + _dot(vn, wsv0) + bs0)
    s = jnp.maximum(so, 0.0)
    gate = jax.nn.sigmoid(_dot(s, wg0) + bg0)
    Vx = _dot(Vhx, wv0) * gate
    Vy = _dot(Vhy, wv0) * gate
    Vz = _dot(Vhz, wv0) * gate

    s, Vx, Vy, Vz = _gvp_block(s, Vx, Vy, Vz, wh1, wss1, wsv1, bs1, wv1, wg1,
                               bg1, True)
    s, Vx, Vy, Vz = _gvp_block(s, Vx, Vy, Vz, wh2, wss2, wsv2, bs2, wv2, wg2,
                               bg2, False)

    os_ref[...] = _dot(pm, s)
    ov_ref[0] = _dot(pm, Vx)
    ov_ref[1] = _dot(pm, Vy)
    ov_ref[2] = _dot(pm, Vz)


def _edge_msg(g, es, ev, hsv, lp):
    m0, m1, m2 = lp['m0'], lp['m1'], lp['m2']
    H0 = 2 * NV + EV
    w = [
        m0['Wh'][:NV], m0['Wh'][NV:NV + 1], m0['Wh'][NV + 1:],
        m0['Ws'][:NS], m0['Ws'][NS:NS + ES], m0['Ws'][NS + ES:2 * NS + ES],
        m0['Ws'][2 * NS + ES:], m0['bs'].reshape(1, NS), m0['Wv'],
        m0['Wg'], m0['bg'].reshape(1, NV),
    ]
    for m in (m1, m2):
        w += [m['Wh'], m['Ws'][:NS], m['Ws'][NS:], m['bs'].reshape(1, NS),
              m['Wv'], m['Wg'], m['bg'].reshape(1, NV)]
    c2 = lambda shp: pl.BlockSpec(shp, lambda i: (0, 0))
    wspecs = [c2(x.shape) for x in w]
    return pl.pallas_call(
        _edge_msg_body,
        grid=(GRID_E,),
        in_specs=[
            pl.BlockSpec((EB, PD), lambda i: (i, 0)),
            pl.BlockSpec((EB, ES), lambda i: (i, 0)),
            pl.BlockSpec((EB, 8), lambda i: (i, 0)),
            pl.BlockSpec((EBN, PD), lambda i: (i, 0)),
        ] + wspecs,
        out_specs=[
            pl.BlockSpec((EBN, NS), lambda i: (i, 0)),
            pl.BlockSpec((3, EBN, NV), lambda i: (0, i, 0)),
        ],
        out_shape=[
            jax.ShapeDtypeStruct((N, NS), _f32),
            jax.ShapeDtypeStruct((3, N, NV), _f32),
        ],
    )(g, es, ev, hsv, *w)


def _node_upd_body(hsv, ags, agv, *rest):
    o_ref = rest[-1]
    (whf0, wssf0, wsvf0, bsf0, wvf0, wgf0, bgf0,
     whf1, wssf1, wsvf1, bsf1, wvf1, wgf1, bgf1) = [r[...] for r in rest[:-1]]
    h = hsv[...]
    s = h[:, :NS] + ags[...]
    Vx = h[:, NS:NS + NV] + agv[0]
    Vy = h[:, NS + NV:NS + 2 * NV] + agv[1]
    Vz = h[:, NS + 2 * NV:NS + 3 * NV] + agv[2]
    s, Vx, Vy, Vz = _ln_sv_block(s, Vx, Vy, Vz)
    fs, fVx, fVy, fVz = _gvp_block(s, Vx, Vy, Vz, whf0, wssf0, wsvf0, bsf0,
                                   wvf0, wgf0, bgf0, True)
    fs, fVx, fVy, fVz = _gvp_block(fs, fVx, fVy, fVz, whf1, wssf1, wsvf1, bsf1,
                                   wvf1, wgf1, bgf1, False)
    s, Vx, Vy, Vz = _ln_sv_block(s + fs, Vx + fVx, Vy + fVy, Vz + fVz)
    o_ref[...] = jnp.concatenate(
        [s, Vx, Vy, Vz, jnp.zeros((s.shape[0], PD - NS - 3 * NV), _f32)],
        axis=1)


def _node_upd(hsv, ags, agv, lp):
    w = []
    for m in (lp['f0'], lp['f1']):
        w += [m['Wh'], m['Ws'][:NS], m['Ws'][NS:], m['bs'].reshape(1, NS),
              m['Wv'], m['Wg'], m['bg'].reshape(1, NV)]
    c2 = lambda shp: pl.BlockSpec(shp, lambda i: (0, 0))
    wspecs = [c2(x.shape) for x in w]
    return pl.pallas_call(
        _node_upd_body,
        grid=(GRID_N,),
        in_specs=[
            pl.BlockSpec((NB, PD), lambda i: (i, 0)),
            pl.BlockSpec((NB, NS), lambda i: (i, 0)),
            pl.BlockSpec((3, NB, NV), lambda i: (0, i, 0)),
        ] + wspecs,
        out_specs=pl.BlockSpec((NB, PD), lambda i: (i, 0)),
        out_shape=jax.ShapeDtypeStruct((N, PD), _f32),
    )(hsv, ags, agv, *w)


def _out_proj_body(hsv, rm, base, msk, *rest):
    out_ref = rest[-1]
    w0, w_0, w_1, w_2 = [r[...] for r in rest[:-1]]
    h = hsv[...]
    acc = _dot(h[:, :NS], w0) + base[...]
    Vx = h[:, NS:NS + NV]
    Vy = h[:, NS + NV:NS + 2 * NV]
    Vz = h[:, NS + 2 * NV:NS + 3 * NV]
    r = rm[...]
    wj = (w_0, w_1, w_2)
    for j in range(3):
        rot = (Vx * r[:, 3 * j:3 * j + 1] + Vy * r[:, 3 * j + 1:3 * j + 2]
               + Vz * r[:, 3 * j + 2:3 * j + 3])
        acc = acc + _dot(rot, wj[j])
    out_ref[...] = acc * msk[:, 0:1]


def _out_proj(hsv, rm, base, msk, w_out):
    w0 = w_out[:NS]
    ws = [w_out[NS + j::3] for j in range(3)]
    c2 = lambda shp: pl.BlockSpec(shp, lambda i: (0, 0))
    return pl.pallas_call(
        _out_proj_body,
        grid=(GRID_N,),
        in_specs=[
            pl.BlockSpec((NB, PD), lambda i: (i, 0)),
            pl.BlockSpec((NB, 16), lambda i: (i, 0)),
            pl.BlockSpec((NB, ED), lambda i: (i, 0)),
            pl.BlockSpec((NB, 8), lambda i: (i, 0)),
            c2(w0.shape), c2(ws[0].shape), c2(ws[1].shape), c2(ws[2].shape),
        ],
        out_specs=pl.BlockSpec((NB, ED), lambda i: (i, 0)),
        out_shape=jax.ShapeDtypeStruct((N, ED), _f32),
    )(hsv, rm, base, msk, w0, *ws)


def _norm_j(v, axis=-1, keepdims=False):
    return jnp.sqrt(jnp.sum(v * v, axis=axis, keepdims=keepdims) + EPS)


def _normalize_j(v, axis=-1):
    return v / _norm_j(v, axis=axis, keepdims=True)


def _gvp_apply_j(p, s, V, activate=True):
    Vh = jnp.einsum('...vi,vh->...hi', V, p['Wh'])
    vn = _norm_j(Vh, axis=-1)
    s_out = jnp.concatenate([s, vn], axis=-1) @ p['Ws'] + p['bs']
    Vo = jnp.einsum('...hi,hv->...vi', Vh, p['Wv'])
    gate = (jax.nn.relu(s_out) if activate else s_out) @ p['Wg'] + p['bg']
    Vo = Vo * jax.nn.sigmoid(gate)[..., None]
    if activate:
        s_out = jax.nn.relu(s_out)
    return s_out, Vo


def _ln_sv_j(s, V):
    mu = jnp.mean(s, axis=-1, keepdims=True)
    var = jnp.var(s, axis=-1, keepdims=True)
    s = (s - mu) / jnp.sqrt(var + 1e-5)
    vn2 = jnp.sum(V * V, axis=-1)
    denom = jnp.sqrt(jnp.mean(vn2, axis=-1, keepdims=True) + EPS)[..., None]
    return s, V / denom


def _ln_j(x):
    mu = jnp.mean(x, axis=-1, keepdims=True)
    var = jnp.var(x, axis=-1, keepdims=True)
    return (x - mu) / jnp.sqrt(var + 1e-5)


def kernel(gt_backbone_pos, single_mask, single_res_rel, aatype, condition_mask, params):
    X = gt_backbone_pos * condition_mask[..., None, None]
    N_, CA, C = X[..., 0, :], X[..., 1, :], X[..., 2, :]
    Xf = X.reshape(B, 3 * L, 3)
    dX = Xf[:, 1:] - Xf[:, :-1]
    U = _normalize_j(dX)
    u2, u1, u0 = U[:, :-2], U[:, 1:-1], U[:, 2:]
    n2 = _normalize_j(jnp.cross(u2, u1))
    n1 = _normalize_j(jnp.cross(u1, u0))
    cosD = jnp.clip(jnp.sum(n2 * n1, -1), -1 + 1e-7, 1 - 1e-7)
    D = jnp.sign(jnp.sum(u2 * n1, -1)) * jnp.arccos(cosD)
    D = jnp.pad(D, ((0, 0), (1, 2)))
    D = D.reshape(B, L, 3)
    dih = jnp.concatenate([jnp.cos(D), jnp.sin(D)], axis=-1)
    v1 = C - CA
    v2 = N_ - CA
    e1 = _normalize_j(v1)
    u2r = v2 - e1 * jnp.sum(e1 * v2, -1, keepdims=True)
    e2 = _normalize_j(u2r)
    e3 = jnp.cross(e1, e2)
    R = jnp.stack([e1, e2, e3], axis=-2)
    fwd = _normalize_j(jnp.pad(CA[:, 1:] - CA[:, :-1], ((0, 0), (0, 1), (0, 0))))
    bwd = _normalize_j(jnp.pad(CA[:, :-1] - CA[:, 1:], ((0, 0), (1, 0), (0, 0))))
    nv_ = _normalize_j(N_ - CA)
    cv_ = _normalize_j(C - CA)
    bis = _normalize_j(nv_ + cv_)
    perp = _normalize_j(jnp.cross(nv_, cv_))
    side = -bis * np.sqrt(1.0 / 3.0) - perp * np.sqrt(2.0 / 3.0)
    node_V = jnp.stack([fwd, bwd, side], axis=-2)
    d2 = jnp.sum((CA[:, :, None, :] - CA[:, None, :, :]) ** 2, -1)
    d2 = d2 + jnp.eye(L, dtype=_f32)[None] * 1e10
    negd, eidx = jax.lax.top_k(-d2, K)
    res_f = single_res_rel.astype(_f32).reshape(N, 1)
    ntab = jnp.concatenate(
        [res_f, CA.reshape(N, 3), jnp.zeros((N, 124), _f32)], axis=1)
    nd = jnp.pad(negd.reshape(E, 1), ((0, 0), (0, 7)))

    hs0, hV0 = _gvp_apply_j(params['ng'], dih, node_V, activate=False)
    hs0, hV0 = _ln_sv_j(hs0, hV0)
    hv0f = hV0.reshape(N, NV, 3)
    hsv = jnp.concatenate(
        [hs0.reshape(N, NS), hv0f[:, :, 0], hv0f[:, :, 1], hv0f[:, :, 2],
         jnp.zeros((N, PD - NS - 3 * NV), _f32)], axis=1)

    offs = (jnp.arange(B, dtype=jnp.int32) * L)[:, None, None]
    src = (eidx.astype(jnp.int32) + offs).reshape(-1)

    gp = _sc_gather(ntab, src, E, 128)
    es_, ev_ = _edge_embed(nd, gp, ntab, params['eg'])

    for l in range(NL):
        lp = params['layers'][l]
        g = _sc_gather(hsv, src, E, PD)
        ags, agv = _edge_msg(g, es_, ev_, hsv, lp)
        hsv = _node_upd(hsv, ags, agv, lp)

    Rt_flat = R.reshape(N, 9)
    rm = jnp.pad(Rt_flat, ((0, 0), (0, 7)))
    rot_in = jnp.einsum('blvi,blij->blvj', node_V, jnp.swapaxes(R, -2, -1))
    in_feat = jnp.concatenate([dih, rot_in.reshape(B, L, 9)], axis=-1)
    comp_in = (in_feat @ params['w_in'] + params['b_in']) * np.sqrt(ED)
    conf = jnp.ones((B, L), _f32)
    mu_c = jnp.linspace(0.0, 1.0, 16)
    rbf_c = jnp.exp(-(((conf[..., None] - mu_c) * 16.0) ** 2))
    comp_conf = rbf_c @ params['w_conf'] + params['b_conf']
    comp_dih = _ln_j(dih @ params['w_dih'] + params['b_dih'])
    comp_aa = jnp.broadcast_to(params['aa_emb'][0], (B, L, ED))
    base = (comp_in + comp_dih + comp_conf + comp_aa
            + params['b_out']).reshape(N, ED)
    msk = jnp.pad(single_mask.reshape(N, 1), ((0, 0), (0, 7)))
    out = _out_proj(hsv, rm, base, msk, params['w_out'])
    return out.reshape(B, L, ED)
